# Initial kernel scaffold; baseline (speedup 1.0000x reference)
import jax
import jax.numpy as jnp
from jax import lax
from jax.experimental import pallas as pl
from jax.experimental.pallas import tpu as pltpu
from jax.experimental.pallas import tpu_sc as plsc

N = 10000
NP = 10240
E = 320000
EP = 327680
NWORK = 32
CHUNKS = 80
CW = 128
RPT = NP // 16
DH = 16
DI = 128
DO = 128
RB = 1024


def _mesh():
    return plsc.VectorSubcoreMesh(
        core_axis_name="c", subcore_axis_name="s", num_cores=2, num_subcores=16
    )


def _zero_rows(ref, nrows):
    z16 = jnp.zeros((16,), jnp.float32)

    def zb(i, _):
        ref[i, :] = z16
        return 0

    lax.fori_loop(0, nrows, zb, 0)


def _deg_body(dst_hbm, out_hbm, dstv, ones_rows, obuf, accum, sem):
    del sem
    cid = lax.axis_index("c")
    sid = lax.axis_index("s")
    wid = cid * 16 + sid

    _zero_rows(obuf, RPT)
    pltpu.sync_copy(obuf, accum.at[pl.ds(sid * RPT, RPT)])

    lane = lax.iota(jnp.int32, 16)
    e0 = jnp.where(lane == 0, 1.0, 0.0).astype(jnp.float32)

    def fill(i, _):
        ones_rows[i, :] = e0
        return 0

    lax.fori_loop(0, CW, fill, 0)

    pltpu.sync_copy(dst_hbm.at[wid], dstv)
    plsc.subcore_barrier()

    def ch(j, _):
        pltpu.sync_copy(ones_rows, accum.at[dstv.at[j]], add=True)
        return 0

    lax.fori_loop(0, CHUNKS, ch, 0)
    plsc.subcore_barrier()

    pltpu.sync_copy(accum.at[pl.ds(sid * RPT, RPT)], obuf)
    pltpu.sync_copy(obuf, out_hbm.at[cid, pl.ds(sid * RPT, RPT)])


_sc_deg = pl.kernel(
    _deg_body,
    out_type=jax.ShapeDtypeStruct((2, NP, DH), jnp.float32),
    mesh=_mesh(),
    scratch_types=[
        pltpu.VMEM((CHUNKS, CW), jnp.int32),
        pltpu.VMEM((CW, DH), jnp.float32),
        pltpu.VMEM((RPT, DH), jnp.float32),
        pltpu.VMEM_SHARED((NP, DH), jnp.float32),
        pltpu.SemaphoreType.DMA,
    ],
)


def _agg_body(g_hbm, src_hbm, dst_hbm, out_hbm, srcv, dstv, rows, obuf, accum, sem):
    cid = lax.axis_index("c")
    sid = lax.axis_index("s")
    wid = cid * 16 + sid

    _zero_rows(obuf, RPT)
    pltpu.sync_copy(obuf, accum.at[pl.ds(sid * RPT, RPT)])

    pltpu.sync_copy(src_hbm.at[wid], srcv)
    pltpu.sync_copy(dst_hbm.at[wid], dstv)
    plsc.subcore_barrier()

    def ch(j, _):
        pltpu.async_copy(g_hbm.at[srcv.at[j]], rows, sem).wait()
        pltpu.sync_copy(rows, accum.at[dstv.at[j]], add=True)
        return 0

    lax.fori_loop(0, CHUNKS, ch, 0)
    plsc.subcore_barrier()

    pltpu.sync_copy(accum.at[pl.ds(sid * RPT, RPT)], obuf)
    pltpu.sync_copy(obuf, out_hbm.at[cid, pl.ds(sid * RPT, RPT)])


_sc_agg = pl.kernel(
    _agg_body,
    out_type=jax.ShapeDtypeStruct((2, NP, DH), jnp.float32),
    mesh=_mesh(),
    scratch_types=[
        pltpu.VMEM((CHUNKS, CW), jnp.int32),
        pltpu.VMEM((CHUNKS, CW), jnp.int32),
        pltpu.VMEM((CW, DH), jnp.float32),
        pltpu.VMEM((RPT, DH), jnp.float32),
        pltpu.VMEM_SHARED((NP, DH), jnp.float32),
        pltpu.SemaphoreType.DMA,
    ],
)


def _tc_a_body(x_ref, w_ref, d0_ref, d1_ref, g_ref, dv_ref):
    deg = 1.0 + d0_ref[:, 0:1] + d1_ref[:, 0:1]
    dinv = lax.rsqrt(deg)
    h = jnp.dot(x_ref[...], w_ref[...], preferred_element_type=jnp.float32)
    g_ref[...] = h * dinv
    dv_ref[...] = jnp.broadcast_to(dinv, dv_ref.shape)


def _tc_a(x_pad, W1, d0, d1):
    return pl.pallas_call(
        _tc_a_body,
        grid=(NP // RB,),
        in_specs=[
            pl.BlockSpec((RB, DI), lambda i: (i, 0)),
            pl.BlockSpec((DI, DH), lambda i: (0, 0)),
            pl.BlockSpec((RB, DH), lambda i: (i, 0)),
            pl.BlockSpec((RB, DH), lambda i: (i, 0)),
        ],
        out_specs=[
            pl.BlockSpec((RB, DH), lambda i: (i, 0)),
            pl.BlockSpec((RB, DH), lambda i: (i, 0)),
        ],
        out_shape=[
            jax.ShapeDtypeStruct((NP, DH), jnp.float32),
            jax.ShapeDtypeStruct((NP, DH), jnp.float32),
        ],
    )(x_pad, W1, d0, d1)


def _tc_b_body(s0_ref, s1_ref, g_ref, dv_ref, b_ref, o_ref):
    dv = dv_ref[...]
    out1 = dv * (s0_ref[...] + s1_ref[...] + g_ref[...]) + b_ref[...]
    o_ref[...] = jnp.maximum(out1, 0.0) * dv


def _tc_b(s0, s1, g1, dv, b1):
    return pl.pallas_call(
        _tc_b_body,
        grid=(NP // RB,),
        in_specs=[
            pl.BlockSpec((RB, DH), lambda i: (i, 0)),
            pl.BlockSpec((RB, DH), lambda i: (i, 0)),
            pl.BlockSpec((RB, DH), lambda i: (i, 0)),
            pl.BlockSpec((RB, DH), lambda i: (i, 0)),
            pl.BlockSpec((1, DH), lambda i: (0, 0)),
        ],
        out_specs=pl.BlockSpec((RB, DH), lambda i: (i, 0)),
        out_shape=jax.ShapeDtypeStruct((NP, DH), jnp.float32),
    )(s0, s1, g1, dv, b1)


def _tc_c_body(s0_ref, s1_ref, g_ref, dv_ref, w_ref, b_ref, o_ref):
    agg = dv_ref[...] * (s0_ref[...] + s1_ref[...] + g_ref[...])
    o_ref[...] = (
        jnp.dot(agg, w_ref[...], preferred_element_type=jnp.float32) + b_ref[...]
    )


def _tc_c(s0, s1, g2, dv, W2, b2):
    return pl.pallas_call(
        _tc_c_body,
        grid=(NP // RB,),
        in_specs=[
            pl.BlockSpec((RB, DH), lambda i: (i, 0)),
            pl.BlockSpec((RB, DH), lambda i: (i, 0)),
            pl.BlockSpec((RB, DH), lambda i: (i, 0)),
            pl.BlockSpec((RB, DH), lambda i: (i, 0)),
            pl.BlockSpec((DH, DO), lambda i: (0, 0)),
            pl.BlockSpec((1, DO), lambda i: (0, 0)),
        ],
        out_specs=pl.BlockSpec((RB, DO), lambda i: (i, 0)),
        out_shape=jax.ShapeDtypeStruct((NP, DO), jnp.float32),
    )(s0, s1, g2, dv, W2, b2)


def kernel(x, edge_index, W1, b1, W2, b2):
    src = edge_index[0]
    dst = edge_index[1]
    pad = jnp.full((EP - E,), N, jnp.int32)
    src_r = jnp.concatenate([src, pad]).reshape(NWORK, CHUNKS, CW)
    dst_r = jnp.concatenate([dst, pad]).reshape(NWORK, CHUNKS, CW)
    x_pad = jnp.pad(x, ((0, NP - N), (0, 0)))

    degp = _sc_deg(dst_r)
    g1, dv = _tc_a(x_pad, W1, degp[0], degp[1])
    s1 = _sc_agg(g1, src_r, dst_r)
    g2 = _tc_b(s1[0], s1[1], g1, dv, b1.reshape(1, DH))
    s2 = _sc_agg(g2, src_r, dst_r)
    out = _tc_c(s2[0], s2[1], g2, dv, W2, b2.reshape(1, DO))
    return out[:N]


# speedup vs baseline: 43.9878x; 43.9878x over previous
import jax
import jax.numpy as jnp
from jax import lax
from jax.experimental import pallas as pl
from jax.experimental.pallas import tpu as pltpu
from jax.experimental.pallas import tpu_sc as plsc

N = 10000
NP = 10240
E = 320000
EP = 327680
NWORK = 32
CHUNKS = 80
CW = 128
RPT = NP // 16
DH = 16
DI = 128
DO = 128
RB = 1024


def _mesh():
    return plsc.VectorSubcoreMesh(
        core_axis_name="c", subcore_axis_name="s", num_cores=2, num_subcores=16
    )


_SC_PARAMS = pltpu.CompilerParams(use_tc_tiling_on_sc=False)


def _zero_rows(ref, nrows):
    z16 = jnp.zeros((16,), jnp.float32)

    def zb(i, _):
        ref[i, :] = z16
        return 0

    lax.fori_loop(0, nrows, zb, 0)


def _deg_body(dst_hbm, out_hbm, dstv, ones_rows, obuf, accum, sem):
    del sem
    cid = lax.axis_index("c")
    sid = lax.axis_index("s")
    wid = cid * 16 + sid

    _zero_rows(obuf, RPT)
    pltpu.sync_copy(obuf, accum.at[pl.ds(sid * RPT, RPT)])

    lane = lax.iota(jnp.int32, 16)
    e0 = jnp.where(lane == 0, 1.0, 0.0).astype(jnp.float32)

    def fill(i, _):
        ones_rows[i, :] = e0
        return 0

    lax.fori_loop(0, CW, fill, 0)

    pltpu.sync_copy(dst_hbm.at[wid], dstv)
    plsc.subcore_barrier()

    def ch(j, _):
        pltpu.sync_copy(ones_rows, accum.at[dstv.at[j]], add=True)
        return 0

    lax.fori_loop(0, CHUNKS, ch, 0)
    plsc.subcore_barrier()

    pltpu.sync_copy(accum.at[pl.ds(sid * RPT, RPT)], obuf)
    pltpu.sync_copy(obuf, out_hbm.at[cid, pl.ds(sid * RPT, RPT)])


_sc_deg = pl.kernel(
    _deg_body,
    out_type=jax.ShapeDtypeStruct((2, NP, DH), jnp.float32),
    mesh=_mesh(),
    scratch_types=[
        pltpu.VMEM((CHUNKS, CW), jnp.int32),
        pltpu.VMEM((CW, DH), jnp.float32),
        pltpu.VMEM((RPT, DH), jnp.float32),
        pltpu.VMEM_SHARED((NP, DH), jnp.float32),
        pltpu.SemaphoreType.DMA,
    ],
    compiler_params=_SC_PARAMS,
)


def _agg_body(g_hbm, src_hbm, dst_hbm, out_hbm, srcv, dstv, rows, obuf, gsh, accum, sem):
    cid = lax.axis_index("c")
    sid = lax.axis_index("s")
    wid = cid * 16 + sid

    _zero_rows(obuf, RPT)
    pltpu.sync_copy(obuf, accum.at[pl.ds(sid * RPT, RPT)])

    pltpu.sync_copy(g_hbm.at[pl.ds(sid * RPT, RPT)], gsh.at[pl.ds(sid * RPT, RPT)])
    pltpu.sync_copy(src_hbm.at[wid], srcv)
    pltpu.sync_copy(dst_hbm.at[wid], dstv)
    plsc.subcore_barrier()

    def ch(j, _):
        pltpu.async_copy(gsh.at[srcv.at[j]], rows, sem).wait()
        pltpu.sync_copy(rows, accum.at[dstv.at[j]], add=True)
        return 0

    lax.fori_loop(0, CHUNKS, ch, 0)
    plsc.subcore_barrier()

    pltpu.sync_copy(accum.at[pl.ds(sid * RPT, RPT)], obuf)
    pltpu.sync_copy(obuf, out_hbm.at[cid, pl.ds(sid * RPT, RPT)])


_sc_agg = pl.kernel(
    _agg_body,
    out_type=jax.ShapeDtypeStruct((2, NP, DH), jnp.float32),
    mesh=_mesh(),
    scratch_types=[
        pltpu.VMEM((CHUNKS, CW), jnp.int32),
        pltpu.VMEM((CHUNKS, CW), jnp.int32),
        pltpu.VMEM((CW, DH), jnp.float32),
        pltpu.VMEM((RPT, DH), jnp.float32),
        pltpu.VMEM_SHARED((NP, DH), jnp.float32),
        pltpu.VMEM_SHARED((NP, DH), jnp.float32),
        pltpu.SemaphoreType.DMA,
    ],
    compiler_params=_SC_PARAMS,
)


def _tc_a_body(x_ref, w_ref, d0_ref, d1_ref, g_ref, dv_ref):
    deg = 1.0 + d0_ref[:, 0:1] + d1_ref[:, 0:1]
    dinv = lax.rsqrt(deg)
    h = jnp.dot(x_ref[...], w_ref[...], preferred_element_type=jnp.float32)
    g_ref[...] = h * dinv
    dv_ref[...] = jnp.broadcast_to(dinv, dv_ref.shape)


def _tc_a(x_pad, W1, d0, d1):
    return pl.pallas_call(
        _tc_a_body,
        grid=(NP // RB,),
        in_specs=[
            pl.BlockSpec((RB, DI), lambda i: (i, 0)),
            pl.BlockSpec((DI, DH), lambda i: (0, 0)),
            pl.BlockSpec((RB, DH), lambda i: (i, 0)),
            pl.BlockSpec((RB, DH), lambda i: (i, 0)),
        ],
        out_specs=[
            pl.BlockSpec((RB, DH), lambda i: (i, 0)),
            pl.BlockSpec((RB, DH), lambda i: (i, 0)),
        ],
        out_shape=[
            jax.ShapeDtypeStruct((NP, DH), jnp.float32),
            jax.ShapeDtypeStruct((NP, DH), jnp.float32),
        ],
    )(x_pad, W1, d0, d1)


def _tc_b_body(s0_ref, s1_ref, g_ref, dv_ref, b_ref, o_ref):
    dv = dv_ref[...]
    out1 = dv * (s0_ref[...] + s1_ref[...] + g_ref[...]) + b_ref[...]
    o_ref[...] = jnp.maximum(out1, 0.0) * dv


def _tc_b(s0, s1, g1, dv, b1):
    return pl.pallas_call(
        _tc_b_body,
        grid=(NP // RB,),
        in_specs=[
            pl.BlockSpec((RB, DH), lambda i: (i, 0)),
            pl.BlockSpec((RB, DH), lambda i: (i, 0)),
            pl.BlockSpec((RB, DH), lambda i: (i, 0)),
            pl.BlockSpec((RB, DH), lambda i: (i, 0)),
            pl.BlockSpec((1, DH), lambda i: (0, 0)),
        ],
        out_specs=pl.BlockSpec((RB, DH), lambda i: (i, 0)),
        out_shape=jax.ShapeDtypeStruct((NP, DH), jnp.float32),
    )(s0, s1, g1, dv, b1)


def _tc_c_body(s0_ref, s1_ref, g_ref, dv_ref, w_ref, b_ref, o_ref):
    agg = dv_ref[...] * (s0_ref[...] + s1_ref[...] + g_ref[...])
    o_ref[...] = (
        jnp.dot(agg, w_ref[...], preferred_element_type=jnp.float32) + b_ref[...]
    )


def _tc_c(s0, s1, g2, dv, W2, b2):
    return pl.pallas_call(
        _tc_c_body,
        grid=(NP // RB,),
        in_specs=[
            pl.BlockSpec((RB, DH), lambda i: (i, 0)),
            pl.BlockSpec((RB, DH), lambda i: (i, 0)),
            pl.BlockSpec((RB, DH), lambda i: (i, 0)),
            pl.BlockSpec((RB, DH), lambda i: (i, 0)),
            pl.BlockSpec((DH, DO), lambda i: (0, 0)),
            pl.BlockSpec((1, DO), lambda i: (0, 0)),
        ],
        out_specs=pl.BlockSpec((RB, DO), lambda i: (i, 0)),
        out_shape=jax.ShapeDtypeStruct((NP, DO), jnp.float32),
    )(s0, s1, g2, dv, W2, b2)


def kernel(x, edge_index, W1, b1, W2, b2):
    src = edge_index[0]
    dst = edge_index[1]
    pad = jnp.full((EP - E,), N, jnp.int32)
    src_r = jnp.concatenate([src, pad]).reshape(NWORK, CHUNKS, CW)
    dst_r = jnp.concatenate([dst, pad]).reshape(NWORK, CHUNKS, CW)
    x_pad = jnp.pad(x, ((0, NP - N), (0, 0)))

    degp = _sc_deg(dst_r)
    g1, dv = _tc_a(x_pad, W1, degp[0], degp[1])
    s1 = _sc_agg(g1, src_r, dst_r)
    g2 = _tc_b(s1[0], s1[1], g1, dv, b1.reshape(1, DH))
    s2 = _sc_agg(g2, src_r, dst_r)
    out = _tc_c(s2[0], s2[1], g2, dv, W2, b2.reshape(1, DO))
    return out[:N]


# speedup vs baseline: 47.2585x; 1.0744x over previous
import jax
import jax.numpy as jnp
from jax import lax
from jax.experimental import pallas as pl
from jax.experimental.pallas import tpu as pltpu
from jax.experimental.pallas import tpu_sc as plsc

N = 10000
NP = 10240
E = 320000
EP = 327680
NWORK = 32
CHUNKS = 80
CW = 128
RPT = NP // 16
DH = 16
DI = 128
DO = 128
RB = 1024


def _mesh():
    return plsc.VectorSubcoreMesh(
        core_axis_name="c", subcore_axis_name="s", num_cores=2, num_subcores=16
    )


_SC_PARAMS = pltpu.CompilerParams(use_tc_tiling_on_sc=False)


def _zero_rows(ref, nrows):
    z16 = jnp.zeros((16,), jnp.float32)

    def zb(i, _):
        ref[i, :] = z16
        return 0

    lax.fori_loop(0, nrows, zb, 0)


def _deg_body(dst_hbm, out_hbm, dstv, ones_rows, obuf, accum, sem):
    cid = lax.axis_index("c")
    sid = lax.axis_index("s")
    wid = cid * 16 + sid

    _zero_rows(obuf, RPT)
    pltpu.sync_copy(obuf, accum.at[pl.ds(sid * RPT, RPT)])

    lane = lax.iota(jnp.int32, 16)
    e0 = jnp.where(lane == 0, 1.0, 0.0).astype(jnp.float32)

    def fill(i, _):
        ones_rows[i, :] = e0
        return 0

    lax.fori_loop(0, CW, fill, 0)

    pltpu.sync_copy(dst_hbm.at[wid], dstv)
    plsc.subcore_barrier()

    def ch(j, _):
        pltpu.async_copy(ones_rows, accum.at[dstv.at[j]], sem, add=True)
        return 0

    lax.fori_loop(0, CHUNKS, ch, 0)

    def dr(j, _):
        pltpu.make_async_copy(ones_rows, accum.at[dstv.at[0]], sem).wait()
        return 0

    lax.fori_loop(0, CHUNKS, dr, 0)
    plsc.subcore_barrier()

    pltpu.sync_copy(accum.at[pl.ds(sid * RPT, RPT)], obuf)
    pltpu.sync_copy(obuf, out_hbm.at[cid, pl.ds(sid * RPT, RPT)])


_sc_deg = pl.kernel(
    _deg_body,
    out_type=jax.ShapeDtypeStruct((2, NP, DH), jnp.float32),
    mesh=_mesh(),
    scratch_types=[
        pltpu.VMEM((CHUNKS, CW), jnp.int32),
        pltpu.VMEM((CW, DH), jnp.float32),
        pltpu.VMEM((RPT, DH), jnp.float32),
        pltpu.VMEM_SHARED((NP, DH), jnp.float32),
        pltpu.SemaphoreType.DMA,
    ],
    compiler_params=_SC_PARAMS,
)


def _agg_body(
    g_hbm, src_hbm, dst_hbm, out_hbm,
    srcv, dstv, rows0, rows1, obuf, gsh, accum,
    sg0, sg1, ss0, ss1,
):
    cid = lax.axis_index("c")
    sid = lax.axis_index("s")
    wid = cid * 16 + sid

    _zero_rows(obuf, RPT)
    pltpu.sync_copy(obuf, accum.at[pl.ds(sid * RPT, RPT)])

    pltpu.sync_copy(g_hbm.at[pl.ds(sid * RPT, RPT)], gsh.at[pl.ds(sid * RPT, RPT)])
    pltpu.sync_copy(src_hbm.at[wid], srcv)
    pltpu.sync_copy(dst_hbm.at[wid], dstv)
    plsc.subcore_barrier()

    pltpu.async_copy(gsh.at[srcv.at[0]], rows0, sg0)

    @pl.loop(0, CHUNKS, step=2)
    def _pipe(j2):
        for b in range(2):
            j = j2 + b
            rb = (rows0, rows1)[b]
            ro = (rows1, rows0)[b]
            sgo = (sg1, sg0)[b]
            sgb = (sg0, sg1)[b]
            sso = (ss1, ss0)[b]
            ssb = (ss0, ss1)[b]

            @pl.when(jnp.logical_and(j + 1 < CHUNKS, j >= 1))
            def _():
                pltpu.make_async_copy(ro, accum.at[dstv.at[0]], sso).wait()

            @pl.when(j + 1 < CHUNKS)
            def _():
                pltpu.async_copy(gsh.at[srcv.at[j + 1]], ro, sgo)

            pltpu.make_async_copy(gsh.at[srcv.at[0]], rb, sgb).wait()
            pltpu.async_copy(rb, accum.at[dstv.at[j]], ssb, add=True)

    pltpu.make_async_copy(rows0, accum.at[dstv.at[0]], ss0).wait()
    pltpu.make_async_copy(rows1, accum.at[dstv.at[0]], ss1).wait()
    plsc.subcore_barrier()

    pltpu.sync_copy(accum.at[pl.ds(sid * RPT, RPT)], obuf)
    pltpu.sync_copy(obuf, out_hbm.at[cid, pl.ds(sid * RPT, RPT)])


_sc_agg = pl.kernel(
    _agg_body,
    out_type=jax.ShapeDtypeStruct((2, NP, DH), jnp.float32),
    mesh=_mesh(),
    scratch_types=[
        pltpu.VMEM((CHUNKS, CW), jnp.int32),
        pltpu.VMEM((CHUNKS, CW), jnp.int32),
        pltpu.VMEM((CW, DH), jnp.float32),
        pltpu.VMEM((CW, DH), jnp.float32),
        pltpu.VMEM((RPT, DH), jnp.float32),
        pltpu.VMEM_SHARED((NP, DH), jnp.float32),
        pltpu.VMEM_SHARED((NP, DH), jnp.float32),
        pltpu.SemaphoreType.DMA,
        pltpu.SemaphoreType.DMA,
        pltpu.SemaphoreType.DMA,
        pltpu.SemaphoreType.DMA,
    ],
    compiler_params=_SC_PARAMS,
)


def _tc_a_body(x_ref, w_ref, d0_ref, d1_ref, g_ref, dv_ref):
    deg = 1.0 + d0_ref[:, 0:1] + d1_ref[:, 0:1]
    dinv = lax.rsqrt(deg)
    h = jnp.dot(x_ref[...], w_ref[...], preferred_element_type=jnp.float32)
    g_ref[...] = h * dinv
    dv_ref[...] = jnp.broadcast_to(dinv, dv_ref.shape)


def _tc_a(x_pad, W1, d0, d1):
    return pl.pallas_call(
        _tc_a_body,
        grid=(NP // RB,),
        in_specs=[
            pl.BlockSpec((RB, DI), lambda i: (i, 0)),
            pl.BlockSpec((DI, DH), lambda i: (0, 0)),
            pl.BlockSpec((RB, DH), lambda i: (i, 0)),
            pl.BlockSpec((RB, DH), lambda i: (i, 0)),
        ],
        out_specs=[
            pl.BlockSpec((RB, DH), lambda i: (i, 0)),
            pl.BlockSpec((RB, DH), lambda i: (i, 0)),
        ],
        out_shape=[
            jax.ShapeDtypeStruct((NP, DH), jnp.float32),
            jax.ShapeDtypeStruct((NP, DH), jnp.float32),
        ],
    )(x_pad, W1, d0, d1)


def _tc_b_body(s0_ref, s1_ref, g_ref, dv_ref, b_ref, o_ref):
    dv = dv_ref[...]
    out1 = dv * (s0_ref[...] + s1_ref[...] + g_ref[...]) + b_ref[...]
    o_ref[...] = jnp.maximum(out1, 0.0) * dv


def _tc_b(s0, s1, g1, dv, b1):
    return pl.pallas_call(
        _tc_b_body,
        grid=(NP // RB,),
        in_specs=[
            pl.BlockSpec((RB, DH), lambda i: (i, 0)),
            pl.BlockSpec((RB, DH), lambda i: (i, 0)),
            pl.BlockSpec((RB, DH), lambda i: (i, 0)),
            pl.BlockSpec((RB, DH), lambda i: (i, 0)),
            pl.BlockSpec((1, DH), lambda i: (0, 0)),
        ],
        out_specs=pl.BlockSpec((RB, DH), lambda i: (i, 0)),
        out_shape=jax.ShapeDtypeStruct((NP, DH), jnp.float32),
    )(s0, s1, g1, dv, b1)


def _tc_c_body(s0_ref, s1_ref, g_ref, dv_ref, w_ref, b_ref, o_ref):
    agg = dv_ref[...] * (s0_ref[...] + s1_ref[...] + g_ref[...])
    o_ref[...] = (
        jnp.dot(agg, w_ref[...], preferred_element_type=jnp.float32) + b_ref[...]
    )


def _tc_c(s0, s1, g2, dv, W2, b2):
    return pl.pallas_call(
        _tc_c_body,
        grid=(NP // RB,),
        in_specs=[
            pl.BlockSpec((RB, DH), lambda i: (i, 0)),
            pl.BlockSpec((RB, DH), lambda i: (i, 0)),
            pl.BlockSpec((RB, DH), lambda i: (i, 0)),
            pl.BlockSpec((RB, DH), lambda i: (i, 0)),
            pl.BlockSpec((DH, DO), lambda i: (0, 0)),
            pl.BlockSpec((1, DO), lambda i: (0, 0)),
        ],
        out_specs=pl.BlockSpec((RB, DO), lambda i: (i, 0)),
        out_shape=jax.ShapeDtypeStruct((NP, DO), jnp.float32),
    )(s0, s1, g2, dv, W2, b2)


def kernel(x, edge_index, W1, b1, W2, b2):
    src = edge_index[0]
    dst = edge_index[1]
    pad = jnp.full((EP - E,), N, jnp.int32)
    src_r = jnp.concatenate([src, pad]).reshape(NWORK, CHUNKS, CW)
    dst_r = jnp.concatenate([dst, pad]).reshape(NWORK, CHUNKS, CW)
    x_pad = jnp.pad(x, ((0, NP - N), (0, 0)))

    degp = _sc_deg(dst_r)
    g1, dv = _tc_a(x_pad, W1, degp[0], degp[1])
    s1 = _sc_agg(g1, src_r, dst_r)
    g2 = _tc_b(s1[0], s1[1], g1, dv, b1.reshape(1, DH))
    s2 = _sc_agg(g2, src_r, dst_r)
    out = _tc_c(s2[0], s2[1], g2, dv, W2, b2.reshape(1, DO))
    return out[:N]


# speedup vs baseline: 50.3492x; 1.0654x over previous
import jax
import jax.numpy as jnp
from jax import lax
from jax.experimental import pallas as pl
from jax.experimental.pallas import tpu as pltpu
from jax.experimental.pallas import tpu_sc as plsc

N = 10000
NP = 10240
E = 320000
EP = 327680
NWORK = 32
CHUNKS = 80
CW = 128
RPT = NP // 16
DH = 16
DI = 128
DO = 128
RB = 1024


def _mesh():
    return plsc.VectorSubcoreMesh(
        core_axis_name="c", subcore_axis_name="s", num_cores=2, num_subcores=16
    )


_SC_PARAMS = pltpu.CompilerParams(use_tc_tiling_on_sc=False)


def _zero_rows(ref, nrows):
    z16 = jnp.zeros((16,), jnp.float32)

    def zb(i, _):
        ref[i, :] = z16
        return 0

    lax.fori_loop(0, nrows, zb, 0)


def _deg_body(dst_hbm, out_hbm, dstv, ones_rows, obuf, accum, sem):
    cid = lax.axis_index("c")
    sid = lax.axis_index("s")
    wid = cid * 16 + sid

    _zero_rows(obuf, RPT)
    pltpu.sync_copy(obuf, accum.at[pl.ds(sid * RPT, RPT)])

    lane = lax.iota(jnp.int32, 16)
    e0 = jnp.where(lane == 0, 1.0, 0.0).astype(jnp.float32)

    def fill(i, _):
        ones_rows[i, :] = e0
        return 0

    lax.fori_loop(0, CW, fill, 0)

    pltpu.sync_copy(dst_hbm.at[wid], dstv)
    plsc.subcore_barrier()

    def ch(j, _):
        pltpu.async_copy(ones_rows, accum.at[dstv.at[j]], sem, add=True)
        return 0

    lax.fori_loop(0, CHUNKS, ch, 0)

    def dr(j, _):
        pltpu.make_async_copy(ones_rows, accum.at[dstv.at[0]], sem).wait()
        return 0

    lax.fori_loop(0, CHUNKS, dr, 0)
    plsc.subcore_barrier()

    pltpu.sync_copy(accum.at[pl.ds(sid * RPT, RPT)], obuf)
    pltpu.sync_copy(obuf, out_hbm.at[cid, pl.ds(sid * RPT, RPT)])


_sc_deg = pl.kernel(
    _deg_body,
    out_type=jax.ShapeDtypeStruct((2, NP, DH), jnp.float32),
    mesh=_mesh(),
    scratch_types=[
        pltpu.VMEM((CHUNKS, CW), jnp.int32),
        pltpu.VMEM((CW, DH), jnp.float32),
        pltpu.VMEM((RPT, DH), jnp.float32),
        pltpu.VMEM_SHARED((NP, DH), jnp.float32),
        pltpu.SemaphoreType.DMA,
    ],
    compiler_params=_SC_PARAMS,
)


def _agg_body(
    g_hbm, src_hbm, dst_hbm, out_hbm,
    srcv, dstv, rows0, rows1, obuf, gsh, accum,
    sg0, sg1, ss0, ss1,
):
    cid = lax.axis_index("c")
    sid = lax.axis_index("s")
    wid = cid * 16 + sid

    _zero_rows(obuf, RPT)
    pltpu.sync_copy(obuf, accum.at[pl.ds(sid * RPT, RPT)])

    pltpu.sync_copy(g_hbm.at[pl.ds(sid * RPT, RPT)], gsh.at[pl.ds(sid * RPT, RPT)])
    pltpu.sync_copy(src_hbm.at[wid], srcv)
    pltpu.sync_copy(dst_hbm.at[wid], dstv)
    plsc.subcore_barrier()

    pltpu.async_copy(gsh.at[srcv.at[0]], rows0, sg0)

    @pl.loop(0, CHUNKS, step=2)
    def _pipe(j2):
        for b in range(2):
            j = j2 + b
            rb = (rows0, rows1)[b]
            ro = (rows1, rows0)[b]
            sgo = (sg1, sg0)[b]
            sgb = (sg0, sg1)[b]
            sso = (ss1, ss0)[b]
            ssb = (ss0, ss1)[b]

            @pl.when(jnp.logical_and(j + 1 < CHUNKS, j >= 1))
            def _():
                pltpu.make_async_copy(ro, accum.at[dstv.at[0]], sso).wait()

            @pl.when(j + 1 < CHUNKS)
            def _():
                pltpu.async_copy(gsh.at[srcv.at[j + 1]], ro, sgo)

            pltpu.make_async_copy(gsh.at[srcv.at[0]], rb, sgb).wait()
            pltpu.async_copy(rb, accum.at[dstv.at[j]], ssb, add=True)

    pltpu.make_async_copy(rows0, accum.at[dstv.at[0]], ss0).wait()
    pltpu.make_async_copy(rows1, accum.at[dstv.at[0]], ss1).wait()
    plsc.subcore_barrier()

    pltpu.sync_copy(accum.at[pl.ds(sid * RPT, RPT)], obuf)
    pltpu.sync_copy(obuf, out_hbm.at[cid, pl.ds(sid * RPT, RPT)])


_sc_agg = pl.kernel(
    _agg_body,
    out_type=jax.ShapeDtypeStruct((2, NP, DH), jnp.float32),
    mesh=_mesh(),
    scratch_types=[
        pltpu.VMEM((CHUNKS, CW), jnp.int32),
        pltpu.VMEM((CHUNKS, CW), jnp.int32),
        pltpu.VMEM((CW, DH), jnp.float32),
        pltpu.VMEM((CW, DH), jnp.float32),
        pltpu.VMEM((RPT, DH), jnp.float32),
        pltpu.VMEM_SHARED((NP, DH), jnp.float32),
        pltpu.VMEM_SHARED((NP, DH), jnp.float32),
        pltpu.SemaphoreType.DMA,
        pltpu.SemaphoreType.DMA,
        pltpu.SemaphoreType.DMA,
        pltpu.SemaphoreType.DMA,
    ],
    compiler_params=_SC_PARAMS,
)


def _tc_a_body(x_ref, w_ref, d0_ref, d1_ref, g_ref, dv_ref):
    deg = 1.0 + d0_ref[:, 0:1] + d1_ref[:, 0:1]
    dinv = lax.rsqrt(deg)
    h = jnp.dot(x_ref[...], w_ref[...], preferred_element_type=jnp.float32)
    g_ref[...] = h * dinv
    dv_ref[...] = jnp.broadcast_to(dinv, dv_ref.shape)


def _tc_a(x_pad, W1, d0, d1):
    return pl.pallas_call(
        _tc_a_body,
        grid=(NP // RB,),
        in_specs=[
            pl.BlockSpec((RB, DI), lambda i: (i, 0)),
            pl.BlockSpec((DI, DH), lambda i: (0, 0)),
            pl.BlockSpec((RB, DH), lambda i: (i, 0)),
            pl.BlockSpec((RB, DH), lambda i: (i, 0)),
        ],
        out_specs=[
            pl.BlockSpec((RB, DH), lambda i: (i, 0)),
            pl.BlockSpec((RB, DH), lambda i: (i, 0)),
        ],
        out_shape=[
            jax.ShapeDtypeStruct((NP, DH), jnp.float32),
            jax.ShapeDtypeStruct((NP, DH), jnp.float32),
        ],
    )(x_pad, W1, d0, d1)


def _tc_b_body(s0_ref, s1_ref, g_ref, dv_ref, b_ref, o_ref):
    dv = dv_ref[...]
    out1 = dv * (s0_ref[...] + s1_ref[...] + g_ref[...]) + b_ref[...]
    o_ref[...] = jnp.maximum(out1, 0.0) * dv


def _tc_b(s0, s1, g1, dv, b1):
    return pl.pallas_call(
        _tc_b_body,
        grid=(NP // RB,),
        in_specs=[
            pl.BlockSpec((RB, DH), lambda i: (i, 0)),
            pl.BlockSpec((RB, DH), lambda i: (i, 0)),
            pl.BlockSpec((RB, DH), lambda i: (i, 0)),
            pl.BlockSpec((RB, DH), lambda i: (i, 0)),
            pl.BlockSpec((1, DH), lambda i: (0, 0)),
        ],
        out_specs=pl.BlockSpec((RB, DH), lambda i: (i, 0)),
        out_shape=jax.ShapeDtypeStruct((NP, DH), jnp.float32),
    )(s0, s1, g1, dv, b1)


def _tc_c_body(s0_ref, s1_ref, g_ref, dv_ref, w_ref, b_ref, o_ref):
    agg = dv_ref[...] * (s0_ref[...] + s1_ref[...] + g_ref[...])
    o_ref[...] = (
        jnp.dot(agg, w_ref[...], preferred_element_type=jnp.float32) + b_ref[...]
    )


def _tc_c(s0, s1, g2, dv, W2, b2):
    return pl.pallas_call(
        _tc_c_body,
        grid=(NP // RB,),
        in_specs=[
            pl.BlockSpec((RB, DH), lambda i: (i, 0)),
            pl.BlockSpec((RB, DH), lambda i: (i, 0)),
            pl.BlockSpec((RB, DH), lambda i: (i, 0)),
            pl.BlockSpec((RB, DH), lambda i: (i, 0)),
            pl.BlockSpec((DH, DO), lambda i: (0, 0)),
            pl.BlockSpec((1, DO), lambda i: (0, 0)),
        ],
        out_specs=pl.BlockSpec((RB, DO), lambda i: (i, 0)),
        out_shape=jax.ShapeDtypeStruct((NP, DO), jnp.float32),
    )(s0, s1, g2, dv, W2, b2)


def kernel(x, edge_index, W1, b1, W2, b2):
    src = edge_index[0]
    dst = edge_index[1]
    pad = N + (jnp.arange(EP - E, dtype=jnp.int32) % (NP - N))
    src_r = jnp.concatenate([src, pad]).reshape(NWORK, CHUNKS, CW)
    dst_r = jnp.concatenate([dst, pad]).reshape(NWORK, CHUNKS, CW)
    x_pad = jnp.pad(x, ((0, NP - N), (0, 0)))

    degp = _sc_deg(dst_r)
    g1, dv = _tc_a(x_pad, W1, degp[0], degp[1])
    s1 = _sc_agg(g1, src_r, dst_r)
    g2 = _tc_b(s1[0], s1[1], g1, dv, b1.reshape(1, DH))
    s2 = _sc_agg(g2, src_r, dst_r)
    out = _tc_c(s2[0], s2[1], g2, dv, W2, b2.reshape(1, DO))
    return out[:N]


# speedup vs baseline: 66.8886x; 1.3285x over previous
import jax
import jax.numpy as jnp
from jax import lax
from jax.experimental import pallas as pl
from jax.experimental.pallas import tpu as pltpu
from jax.experimental.pallas import tpu_sc as plsc

N = 10000
NP = 10240
E = 320000
EP = 327680
NWORK = 32
CHUNKS = 80
CW = 128
RPT = NP // 16
DH = 16
DI = 128
DO = 128
RB = 1024


def _mesh():
    return plsc.VectorSubcoreMesh(
        core_axis_name="c", subcore_axis_name="s", num_cores=2, num_subcores=16
    )


_SC_PARAMS = pltpu.CompilerParams(use_tc_tiling_on_sc=False)


def _rsqrt16(x):
    xi = lax.bitcast_convert_type(x, jnp.int32)
    yi = jnp.int32(0x5F3759DF) - (xi >> 1)
    y = lax.bitcast_convert_type(yi, jnp.float32)
    for _ in range(3):
        y = y * (1.5 - 0.5 * x * y * y)
    return y


def _zero_rows(ref, nrows):
    z16 = jnp.zeros((16,), jnp.float32)

    def zb(i, _):
        ref[i, :] = z16
        return 0

    lax.fori_loop(0, nrows, zb, 0)


def _edge_pipeline(srcv, dstv, rows0, rows1, gsh, accum, sg0, sg1, ss0, ss1):
    pltpu.async_copy(gsh.at[srcv.at[0]], rows0, sg0)

    @pl.loop(0, CHUNKS, step=2)
    def _pipe(j2):
        for b in range(2):
            j = j2 + b
            rb = (rows0, rows1)[b]
            ro = (rows1, rows0)[b]
            sgo = (sg1, sg0)[b]
            sgb = (sg0, sg1)[b]
            sso = (ss1, ss0)[b]
            ssb = (ss0, ss1)[b]

            @pl.when(jnp.logical_and(j + 1 < CHUNKS, j >= 1))
            def _():
                pltpu.make_async_copy(ro, accum.at[dstv.at[0]], sso).wait()

            @pl.when(j + 1 < CHUNKS)
            def _():
                pltpu.async_copy(gsh.at[srcv.at[j + 1]], ro, sgo)

            pltpu.make_async_copy(gsh.at[srcv.at[0]], rb, sgb).wait()
            pltpu.async_copy(rb, accum.at[dstv.at[j]], ssb, add=True)

    pltpu.make_async_copy(rows0, accum.at[dstv.at[0]], ss0).wait()
    pltpu.make_async_copy(rows1, accum.at[dstv.at[0]], ss1).wait()


def _deg_body(dst_hbm, out_hbm, dstv, ones_rows, obuf, accum, sem):
    cid = lax.axis_index("c")
    sid = lax.axis_index("s")
    wid = cid * 16 + sid

    _zero_rows(obuf, RPT)
    pltpu.sync_copy(obuf, accum.at[pl.ds(sid * RPT, RPT)])

    one = jnp.ones((16,), jnp.float32)

    def fill(i, _):
        ones_rows[i, :] = one
        return 0

    lax.fori_loop(0, CW, fill, 0)

    pltpu.sync_copy(dst_hbm.at[wid], dstv)
    plsc.subcore_barrier()

    def ch(j, _):
        pltpu.async_copy(ones_rows, accum.at[dstv.at[j]], sem, add=True)
        return 0

    lax.fori_loop(0, CHUNKS, ch, 0)

    def dr(j, _):
        pltpu.make_async_copy(ones_rows, accum.at[dstv.at[0]], sem).wait()
        return 0

    lax.fori_loop(0, CHUNKS, dr, 0)
    plsc.subcore_barrier()

    pltpu.sync_copy(accum.at[pl.ds(sid * RPT, RPT)], obuf)
    pltpu.sync_copy(obuf, out_hbm.at[cid, pl.ds(sid * RPT, RPT)])


_sc_deg = pl.kernel(
    _deg_body,
    out_type=jax.ShapeDtypeStruct((2, NP, DH), jnp.float32),
    mesh=_mesh(),
    scratch_types=[
        pltpu.VMEM((CHUNKS, CW), jnp.int32),
        pltpu.VMEM((CW, DH), jnp.float32),
        pltpu.VMEM((RPT, DH), jnp.float32),
        pltpu.VMEM_SHARED((NP, DH), jnp.float32),
        pltpu.SemaphoreType.DMA,
    ],
    compiler_params=_SC_PARAMS,
)


def _agg1_body(
    h_hbm, d_hbm, src_hbm, dst_hbm,
    s1_hbm, g1_hbm, dv_hbm,
    srcv, dstv, rows0, rows1, obuf, vh, vd0, vd1, gsh, accum,
    sg0, sg1, ss0, ss1,
):
    cid = lax.axis_index("c")
    sid = lax.axis_index("s")
    wid = cid * 16 + sid
    sl = pl.ds(sid * RPT, RPT)

    pltpu.async_copy(h_hbm.at[sl], vh, sg0)
    pltpu.async_copy(d_hbm.at[0, sl], vd0, sg1)
    pltpu.async_copy(d_hbm.at[1, sl], vd1, ss0)
    _zero_rows(obuf, RPT)
    pltpu.sync_copy(obuf, accum.at[sl])
    pltpu.sync_copy(src_hbm.at[wid], srcv)
    pltpu.sync_copy(dst_hbm.at[wid], dstv)
    pltpu.make_async_copy(h_hbm.at[sl], vh, sg0).wait()
    pltpu.make_async_copy(d_hbm.at[0, sl], vd0, sg1).wait()
    pltpu.make_async_copy(d_hbm.at[1, sl], vd1, ss0).wait()

    def cb(i, _):
        deg = 1.0 + vd0[i, :] + vd1[i, :]
        dv = _rsqrt16(deg)
        vh[i, :] = vh[i, :] * dv
        vd1[i, :] = dv
        return 0

    lax.fori_loop(0, RPT, cb, 0, unroll=2)

    pltpu.sync_copy(vh, gsh.at[sl])
    pltpu.sync_copy(vh, g1_hbm.at[sl])
    pltpu.sync_copy(vd1, dv_hbm.at[sl])
    plsc.subcore_barrier()
    _edge_pipeline(srcv, dstv, rows0, rows1, gsh, accum, sg0, sg1, ss0, ss1)
    plsc.subcore_barrier()
    pltpu.sync_copy(accum.at[sl], obuf)
    pltpu.sync_copy(obuf, s1_hbm.at[cid, sl])


_sc_agg1 = pl.kernel(
    _agg1_body,
    out_type=[
        jax.ShapeDtypeStruct((2, NP, DH), jnp.float32),
        jax.ShapeDtypeStruct((NP, DH), jnp.float32),
        jax.ShapeDtypeStruct((NP, DH), jnp.float32),
    ],
    mesh=_mesh(),
    scratch_types=[
        pltpu.VMEM((CHUNKS, CW), jnp.int32),
        pltpu.VMEM((CHUNKS, CW), jnp.int32),
        pltpu.VMEM((CW, DH), jnp.float32),
        pltpu.VMEM((CW, DH), jnp.float32),
        pltpu.VMEM((RPT, DH), jnp.float32),
        pltpu.VMEM((RPT, DH), jnp.float32),
        pltpu.VMEM((RPT, DH), jnp.float32),
        pltpu.VMEM((RPT, DH), jnp.float32),
        pltpu.VMEM_SHARED((NP, DH), jnp.float32),
        pltpu.VMEM_SHARED((NP, DH), jnp.float32),
        pltpu.SemaphoreType.DMA,
        pltpu.SemaphoreType.DMA,
        pltpu.SemaphoreType.DMA,
        pltpu.SemaphoreType.DMA,
    ],
    compiler_params=_SC_PARAMS,
)


def _agg2_body(
    s1_hbm, g1_hbm, dv_hbm, b1_hbm, src_hbm, dst_hbm,
    s2_hbm, g2_hbm,
    srcv, dstv, rows0, rows1, obuf, vg, vs0, vs1, vdv, b1v, gsh, accum,
    sg0, sg1, ss0, ss1,
):
    cid = lax.axis_index("c")
    sid = lax.axis_index("s")
    wid = cid * 16 + sid
    sl = pl.ds(sid * RPT, RPT)

    pltpu.async_copy(g1_hbm.at[sl], vg, sg0)
    pltpu.async_copy(s1_hbm.at[0, sl], vs0, sg1)
    pltpu.async_copy(s1_hbm.at[1, sl], vs1, ss0)
    pltpu.async_copy(dv_hbm.at[sl], vdv, ss1)
    _zero_rows(obuf, RPT)
    pltpu.sync_copy(obuf, accum.at[sl])
    pltpu.sync_copy(src_hbm.at[wid], srcv)
    pltpu.sync_copy(dst_hbm.at[wid], dstv)
    pltpu.sync_copy(b1_hbm, b1v)
    b1 = b1v[...]
    pltpu.make_async_copy(g1_hbm.at[sl], vg, sg0).wait()
    pltpu.make_async_copy(s1_hbm.at[0, sl], vs0, sg1).wait()
    pltpu.make_async_copy(s1_hbm.at[1, sl], vs1, ss0).wait()
    pltpu.make_async_copy(dv_hbm.at[sl], vdv, ss1).wait()

    def cb(i, _):
        dv = vdv[i, :]
        o1 = dv * (vs0[i, :] + vs1[i, :] + vg[i, :]) + b1
        vg[i, :] = jnp.maximum(o1, 0.0) * dv
        return 0

    lax.fori_loop(0, RPT, cb, 0, unroll=2)

    pltpu.sync_copy(vg, gsh.at[sl])
    pltpu.sync_copy(vg, g2_hbm.at[sl])
    plsc.subcore_barrier()
    _edge_pipeline(srcv, dstv, rows0, rows1, gsh, accum, sg0, sg1, ss0, ss1)
    plsc.subcore_barrier()
    pltpu.sync_copy(accum.at[sl], obuf)
    pltpu.sync_copy(obuf, s2_hbm.at[cid, sl])


_sc_agg2 = pl.kernel(
    _agg2_body,
    out_type=[
        jax.ShapeDtypeStruct((2, NP, DH), jnp.float32),
        jax.ShapeDtypeStruct((NP, DH), jnp.float32),
    ],
    mesh=_mesh(),
    scratch_types=[
        pltpu.VMEM((CHUNKS, CW), jnp.int32),
        pltpu.VMEM((CHUNKS, CW), jnp.int32),
        pltpu.VMEM((CW, DH), jnp.float32),
        pltpu.VMEM((CW, DH), jnp.float32),
        pltpu.VMEM((RPT, DH), jnp.float32),
        pltpu.VMEM((RPT, DH), jnp.float32),
        pltpu.VMEM((RPT, DH), jnp.float32),
        pltpu.VMEM((RPT, DH), jnp.float32),
        pltpu.VMEM((RPT, DH), jnp.float32),
        pltpu.VMEM((DH,), jnp.float32),
        pltpu.VMEM_SHARED((NP, DH), jnp.float32),
        pltpu.VMEM_SHARED((NP, DH), jnp.float32),
        pltpu.SemaphoreType.DMA,
        pltpu.SemaphoreType.DMA,
        pltpu.SemaphoreType.DMA,
        pltpu.SemaphoreType.DMA,
    ],
    compiler_params=_SC_PARAMS,
)


def _fin_body(s2_hbm, g2_hbm, dv_hbm, aggf_hbm, vs0, vs1, vg, vdv, sg0, sg1, ss0, ss1):
    sid = lax.axis_index("s")
    cid = lax.axis_index("c")
    w = sid * 2 + cid
    sl = pl.ds(w * (RPT // 2), RPT // 2)

    pltpu.async_copy(s2_hbm.at[0, sl], vs0, sg0)
    pltpu.async_copy(s2_hbm.at[1, sl], vs1, sg1)
    pltpu.async_copy(g2_hbm.at[sl], vg, ss0)
    pltpu.async_copy(dv_hbm.at[sl], vdv, ss1)
    pltpu.make_async_copy(s2_hbm.at[0, sl], vs0, sg0).wait()
    pltpu.make_async_copy(s2_hbm.at[1, sl], vs1, sg1).wait()
    pltpu.make_async_copy(g2_hbm.at[sl], vg, ss0).wait()
    pltpu.make_async_copy(dv_hbm.at[sl], vdv, ss1).wait()

    def cb(i, _):
        vg[i, :] = vdv[i, :] * (vs0[i, :] + vs1[i, :] + vg[i, :])
        return 0

    lax.fori_loop(0, RPT // 2, cb, 0, unroll=2)
    pltpu.sync_copy(vg, aggf_hbm.at[sl])


_sc_fin = pl.kernel(
    _fin_body,
    out_type=jax.ShapeDtypeStruct((NP, DH), jnp.float32),
    mesh=_mesh(),
    scratch_types=[
        pltpu.VMEM((RPT // 2, DH), jnp.float32),
        pltpu.VMEM((RPT // 2, DH), jnp.float32),
        pltpu.VMEM((RPT // 2, DH), jnp.float32),
        pltpu.VMEM((RPT // 2, DH), jnp.float32),
        pltpu.SemaphoreType.DMA,
        pltpu.SemaphoreType.DMA,
        pltpu.SemaphoreType.DMA,
        pltpu.SemaphoreType.DMA,
    ],
    compiler_params=_SC_PARAMS,
)


def _tc_a_body(x_ref, w_ref, h_ref):
    h_ref[...] = jnp.dot(x_ref[...], w_ref[...], preferred_element_type=jnp.float32)


def _tc_a(x, W1):
    return pl.pallas_call(
        _tc_a_body,
        grid=(NP // RB,),
        in_specs=[
            pl.BlockSpec((RB, DI), lambda i: (i, 0)),
            pl.BlockSpec((DI, DH), lambda i: (0, 0)),
        ],
        out_specs=pl.BlockSpec((RB, DH), lambda i: (i, 0)),
        out_shape=jax.ShapeDtypeStruct((NP, DH), jnp.float32),
    )(x, W1)


def _tc_c_body(a_ref, w_ref, b_ref, o_ref):
    o_ref[...] = (
        jnp.dot(a_ref[...], w_ref[...], preferred_element_type=jnp.float32)
        + b_ref[...]
    )


def _tc_c(aggf, W2, b2):
    return pl.pallas_call(
        _tc_c_body,
        grid=(NP // RB,),
        in_specs=[
            pl.BlockSpec((RB, DH), lambda i: (i, 0)),
            pl.BlockSpec((DH, DO), lambda i: (0, 0)),
            pl.BlockSpec((1, DO), lambda i: (0, 0)),
        ],
        out_specs=pl.BlockSpec((RB, DO), lambda i: (i, 0)),
        out_shape=jax.ShapeDtypeStruct((N, DO), jnp.float32),
    )(aggf, W2, b2)


def kernel(x, edge_index, W1, b1, W2, b2):
    src = edge_index[0]
    dst = edge_index[1]
    pad = N + (jnp.arange(EP - E, dtype=jnp.int32) % (NP - N))
    src_r = jnp.concatenate([src, pad]).reshape(NWORK, CHUNKS, CW)
    dst_r = jnp.concatenate([dst, pad]).reshape(NWORK, CHUNKS, CW)

    h = _tc_a(x, W1)
    degp = _sc_deg(dst_r)
    s1, g1, dv = _sc_agg1(h, degp, src_r, dst_r)
    s2, g2 = _sc_agg2(s1, g1, dv, b1, src_r, dst_r)
    aggf = _sc_fin(s2, g2, dv)
    return _tc_c(aggf, W2, b2.reshape(1, DO))


# speedup vs baseline: 72.4520x; 1.0832x over previous
import numpy as np

import jax
import jax.numpy as jnp
from jax import lax
from jax.experimental import pallas as pl
from jax.experimental.pallas import tpu as pltpu
from jax.experimental.pallas import tpu_sc as plsc

N = 10000
NP = 10240
E = 320000
EP = 327680
NWORK = 32
CHUNKS = 80
CW = 128
RPT = NP // 16
DH = 16
DI = 128
DO = 128
RB = 1024

_PAD_EDGES = np.stack([
    np.zeros(EP - E, np.int32),
    (N + np.arange(EP - E) % (NP - N)).astype(np.int32),
])


def _mesh():
    return plsc.VectorSubcoreMesh(
        core_axis_name="c", subcore_axis_name="s", num_cores=2, num_subcores=16
    )


_SC_PARAMS = pltpu.CompilerParams(use_tc_tiling_on_sc=False)


def _rsqrt16(x):
    xi = lax.bitcast_convert_type(x, jnp.int32)
    yi = jnp.int32(0x5F3759DF) - (xi >> 1)
    y = lax.bitcast_convert_type(yi, jnp.float32)
    for _ in range(3):
        y = y * (1.5 - 0.5 * x * y * y)
    return y


def _edge_pipeline(srcv, dstv, bufs, gsh, accum, sg, ss):
    nb = len(bufs)
    pltpu.async_copy(gsh.at[srcv.at[0]], bufs[0], sg[0])
    pltpu.async_copy(gsh.at[srcv.at[1]], bufs[1], sg[1])

    @pl.loop(0, CHUNKS, step=nb)
    def _pipe(j0):
        for b in range(nb):
            j = j0 + b
            b2 = (b + 2) % nb

            @pl.when(jnp.logical_and(j + 2 < CHUNKS, j >= 2))
            def _():
                pltpu.make_async_copy(bufs[b2], accum.at[dstv.at[0]], ss[b2]).wait()

            @pl.when(j + 2 < CHUNKS)
            def _():
                pltpu.async_copy(gsh.at[srcv.at[j + 2]], bufs[b2], sg[b2])

            pltpu.make_async_copy(gsh.at[srcv.at[0]], bufs[b], sg[b]).wait()
            pltpu.async_copy(bufs[b], accum.at[dstv.at[j]], ss[b], add=True)

    for b in range(nb):
        pltpu.make_async_copy(bufs[b], accum.at[dstv.at[0]], ss[b]).wait()


def _deg_body(er_hbm, ones_hbm, zeros_hbm, out_hbm, dstv, ones_rows, accum, sem, sem2):
    cid = lax.axis_index("c")
    sid = lax.axis_index("s")
    wid = cid * 16 + sid
    sl = pl.ds(sid * RPT, RPT)

    pltpu.async_copy(ones_hbm, ones_rows, sem2)
    pltpu.sync_copy(zeros_hbm, accum.at[sl])
    pltpu.sync_copy(er_hbm.at[1, wid], dstv)
    pltpu.make_async_copy(ones_hbm, ones_rows, sem2).wait()
    plsc.subcore_barrier()

    def ch(j, _):
        pltpu.async_copy(ones_rows, accum.at[dstv.at[j]], sem, add=True)
        return 0

    lax.fori_loop(0, CHUNKS, ch, 0)

    def dr(j, _):
        pltpu.make_async_copy(ones_rows, accum.at[dstv.at[0]], sem).wait()
        return 0

    lax.fori_loop(0, CHUNKS, dr, 0)
    plsc.subcore_barrier()

    pltpu.sync_copy(accum.at[sl], out_hbm.at[cid, sl])


_sc_deg = pl.kernel(
    _deg_body,
    out_type=jax.ShapeDtypeStruct((2, NP, DH), jnp.float32),
    mesh=_mesh(),
    scratch_types=[
        pltpu.VMEM((CHUNKS, CW), jnp.int32),
        pltpu.VMEM((CW, DH), jnp.float32),
        pltpu.VMEM_SHARED((NP, DH), jnp.float32),
        pltpu.SemaphoreType.DMA,
        pltpu.SemaphoreType.DMA,
    ],
    compiler_params=_SC_PARAMS,
)


def _agg1_body(
    h_hbm, d_hbm, er_hbm, zeros_hbm,
    s1_hbm, g1_hbm, dv_hbm,
    srcv, dstv, r0, r1, r2, r3, vh, vd0, vd1, gsh, accum,
    g0, g1s, g2s, g3, s0, s1s, s2s, s3,
):
    cid = lax.axis_index("c")
    sid = lax.axis_index("s")
    wid = cid * 16 + sid
    sl = pl.ds(sid * RPT, RPT)

    pltpu.async_copy(h_hbm.at[sl], vh, g0)
    pltpu.async_copy(d_hbm.at[0, sl], vd0, g1s)
    pltpu.async_copy(d_hbm.at[1, sl], vd1, g2s)
    pltpu.sync_copy(zeros_hbm, accum.at[sl])
    pltpu.sync_copy(er_hbm.at[0, wid], srcv)
    pltpu.sync_copy(er_hbm.at[1, wid], dstv)
    pltpu.make_async_copy(h_hbm.at[sl], vh, g0).wait()
    pltpu.make_async_copy(d_hbm.at[0, sl], vd0, g1s).wait()
    pltpu.make_async_copy(d_hbm.at[1, sl], vd1, g2s).wait()

    def cb(i, _):
        deg = 1.0 + vd0[i, :] + vd1[i, :]
        dv = _rsqrt16(deg)
        vh[i, :] = vh[i, :] * dv
        vd1[i, :] = dv
        return 0

    lax.fori_loop(0, RPT, cb, 0, unroll=2)

    pltpu.sync_copy(vh, gsh.at[sl])
    pltpu.async_copy(vh, g1_hbm.at[sl], g0)
    pltpu.async_copy(vd1, dv_hbm.at[sl], g1s)
    pltpu.make_async_copy(vh, g1_hbm.at[sl], g0).wait()
    pltpu.make_async_copy(vd1, dv_hbm.at[sl], g1s).wait()
    plsc.subcore_barrier()
    _edge_pipeline(srcv, dstv, (r0, r1, r2, r3), gsh, accum,
                   (g0, g1s, g2s, g3), (s0, s1s, s2s, s3))
    plsc.subcore_barrier()
    pltpu.sync_copy(accum.at[sl], s1_hbm.at[cid, sl])


_sc_agg1 = pl.kernel(
    _agg1_body,
    out_type=[
        jax.ShapeDtypeStruct((2, NP, DH), jnp.float32),
        jax.ShapeDtypeStruct((NP, DH), jnp.float32),
        jax.ShapeDtypeStruct((NP, DH), jnp.float32),
    ],
    mesh=_mesh(),
    scratch_types=[
        pltpu.VMEM((CHUNKS, CW), jnp.int32),
        pltpu.VMEM((CHUNKS, CW), jnp.int32),
        pltpu.VMEM((CW, DH), jnp.float32),
        pltpu.VMEM((CW, DH), jnp.float32),
        pltpu.VMEM((CW, DH), jnp.float32),
        pltpu.VMEM((CW, DH), jnp.float32),
        pltpu.VMEM((RPT, DH), jnp.float32),
        pltpu.VMEM((RPT, DH), jnp.float32),
        pltpu.VMEM((RPT, DH), jnp.float32),
        pltpu.VMEM_SHARED((NP, DH), jnp.float32),
        pltpu.VMEM_SHARED((NP, DH), jnp.float32),
        pltpu.SemaphoreType.DMA,
        pltpu.SemaphoreType.DMA,
        pltpu.SemaphoreType.DMA,
        pltpu.SemaphoreType.DMA,
        pltpu.SemaphoreType.DMA,
        pltpu.SemaphoreType.DMA,
        pltpu.SemaphoreType.DMA,
        pltpu.SemaphoreType.DMA,
    ],
    compiler_params=_SC_PARAMS,
)


def _agg2_body(
    s1_hbm, g1_hbm, dv_hbm, b1_hbm, er_hbm, zeros_hbm,
    s2_hbm, g2_hbm,
    srcv, dstv, r0, r1, r2, r3, vg, vs0, vs1, vdv, b1v, gsh, accum,
    g0, g1s, g2s, g3, s0, s1s, s2s, s3,
):
    cid = lax.axis_index("c")
    sid = lax.axis_index("s")
    wid = cid * 16 + sid
    sl = pl.ds(sid * RPT, RPT)

    pltpu.async_copy(g1_hbm.at[sl], vg, g0)
    pltpu.async_copy(s1_hbm.at[0, sl], vs0, g1s)
    pltpu.async_copy(s1_hbm.at[1, sl], vs1, g2s)
    pltpu.async_copy(dv_hbm.at[sl], vdv, g3)
    pltpu.sync_copy(zeros_hbm, accum.at[sl])
    pltpu.sync_copy(er_hbm.at[0, wid], srcv)
    pltpu.sync_copy(er_hbm.at[1, wid], dstv)
    pltpu.sync_copy(b1_hbm, b1v)
    b1 = b1v[...]
    pltpu.make_async_copy(g1_hbm.at[sl], vg, g0).wait()
    pltpu.make_async_copy(s1_hbm.at[0, sl], vs0, g1s).wait()
    pltpu.make_async_copy(s1_hbm.at[1, sl], vs1, g2s).wait()
    pltpu.make_async_copy(dv_hbm.at[sl], vdv, g3).wait()

    def cb(i, _):
        dv = vdv[i, :]
        o1 = dv * (vs0[i, :] + vs1[i, :] + vg[i, :]) + b1
        vg[i, :] = jnp.maximum(o1, 0.0) * dv
        return 0

    lax.fori_loop(0, RPT, cb, 0, unroll=2)

    pltpu.sync_copy(vg, gsh.at[sl])
    pltpu.async_copy(vg, g2_hbm.at[sl], g0)
    pltpu.make_async_copy(vg, g2_hbm.at[sl], g0).wait()
    plsc.subcore_barrier()
    _edge_pipeline(srcv, dstv, (r0, r1, r2, r3), gsh, accum,
                   (g0, g1s, g2s, g3), (s0, s1s, s2s, s3))
    plsc.subcore_barrier()
    pltpu.sync_copy(accum.at[sl], s2_hbm.at[cid, sl])


_sc_agg2 = pl.kernel(
    _agg2_body,
    out_type=[
        jax.ShapeDtypeStruct((2, NP, DH), jnp.float32),
        jax.ShapeDtypeStruct((NP, DH), jnp.float32),
    ],
    mesh=_mesh(),
    scratch_types=[
        pltpu.VMEM((CHUNKS, CW), jnp.int32),
        pltpu.VMEM((CHUNKS, CW), jnp.int32),
        pltpu.VMEM((CW, DH), jnp.float32),
        pltpu.VMEM((CW, DH), jnp.float32),
        pltpu.VMEM((CW, DH), jnp.float32),
        pltpu.VMEM((CW, DH), jnp.float32),
        pltpu.VMEM((RPT, DH), jnp.float32),
        pltpu.VMEM((RPT, DH), jnp.float32),
        pltpu.VMEM((RPT, DH), jnp.float32),
        pltpu.VMEM((RPT, DH), jnp.float32),
        pltpu.VMEM((DH,), jnp.float32),
        pltpu.VMEM_SHARED((NP, DH), jnp.float32),
        pltpu.VMEM_SHARED((NP, DH), jnp.float32),
        pltpu.SemaphoreType.DMA,
        pltpu.SemaphoreType.DMA,
        pltpu.SemaphoreType.DMA,
        pltpu.SemaphoreType.DMA,
        pltpu.SemaphoreType.DMA,
        pltpu.SemaphoreType.DMA,
        pltpu.SemaphoreType.DMA,
        pltpu.SemaphoreType.DMA,
    ],
    compiler_params=_SC_PARAMS,
)


def _fin_body(s2_hbm, g2_hbm, dv_hbm, aggf_hbm, vs0, vs1, vg, vdv, g0, g1s, g2s, g3):
    sid = lax.axis_index("s")
    cid = lax.axis_index("c")
    w = sid * 2 + cid
    sl = pl.ds(w * (RPT // 2), RPT // 2)

    pltpu.async_copy(s2_hbm.at[0, sl], vs0, g0)
    pltpu.async_copy(s2_hbm.at[1, sl], vs1, g1s)
    pltpu.async_copy(g2_hbm.at[sl], vg, g2s)
    pltpu.async_copy(dv_hbm.at[sl], vdv, g3)
    pltpu.make_async_copy(s2_hbm.at[0, sl], vs0, g0).wait()
    pltpu.make_async_copy(s2_hbm.at[1, sl], vs1, g1s).wait()
    pltpu.make_async_copy(g2_hbm.at[sl], vg, g2s).wait()
    pltpu.make_async_copy(dv_hbm.at[sl], vdv, g3).wait()

    def cb(i, _):
        vg[i, :] = vdv[i, :] * (vs0[i, :] + vs1[i, :] + vg[i, :])
        return 0

    lax.fori_loop(0, RPT // 2, cb, 0, unroll=2)
    pltpu.sync_copy(vg, aggf_hbm.at[sl])


_sc_fin = pl.kernel(
    _fin_body,
    out_type=jax.ShapeDtypeStruct((NP, DH), jnp.float32),
    mesh=_mesh(),
    scratch_types=[
        pltpu.VMEM((RPT // 2, DH), jnp.float32),
        pltpu.VMEM((RPT // 2, DH), jnp.float32),
        pltpu.VMEM((RPT // 2, DH), jnp.float32),
        pltpu.VMEM((RPT // 2, DH), jnp.float32),
        pltpu.SemaphoreType.DMA,
        pltpu.SemaphoreType.DMA,
        pltpu.SemaphoreType.DMA,
        pltpu.SemaphoreType.DMA,
    ],
    compiler_params=_SC_PARAMS,
)


def _tc_a_body(x_ref, w_ref, h_ref):
    h_ref[...] = jnp.dot(x_ref[...], w_ref[...], preferred_element_type=jnp.float32)


def _tc_a(x, W1):
    return pl.pallas_call(
        _tc_a_body,
        grid=(NP // RB,),
        in_specs=[
            pl.BlockSpec((RB, DI), lambda i: (i, 0)),
            pl.BlockSpec((DI, DH), lambda i: (0, 0)),
        ],
        out_specs=pl.BlockSpec((RB, DH), lambda i: (i, 0)),
        out_shape=jax.ShapeDtypeStruct((NP, DH), jnp.float32),
    )(x, W1)


def _tc_c_body(a_ref, w_ref, b_ref, o_ref):
    o_ref[...] = (
        jnp.dot(a_ref[...], w_ref[...], preferred_element_type=jnp.float32)
        + b_ref[...]
    )


def _tc_c(aggf, W2, b2):
    return pl.pallas_call(
        _tc_c_body,
        grid=(NP // RB,),
        in_specs=[
            pl.BlockSpec((RB, DH), lambda i: (i, 0)),
            pl.BlockSpec((DH, DO), lambda i: (0, 0)),
            pl.BlockSpec((1, DO), lambda i: (0, 0)),
        ],
        out_specs=pl.BlockSpec((RB, DO), lambda i: (i, 0)),
        out_shape=jax.ShapeDtypeStruct((N, DO), jnp.float32),
    )(aggf, W2, b2)


def kernel(x, edge_index, W1, b1, W2, b2):
    er = jnp.concatenate(
        [edge_index, jnp.asarray(_PAD_EDGES)], axis=1
    ).reshape(2, NWORK, CHUNKS, CW)
    zeros_c = jnp.zeros((RPT, DH), jnp.float32)
    ones_c = jnp.ones((CW, DH), jnp.float32)

    h = _tc_a(x, W1)
    degp = _sc_deg(er, ones_c, zeros_c)
    s1, g1, dv = _sc_agg1(h, degp, er, zeros_c)
    s2, g2 = _sc_agg2(s1, g1, dv, b1, er, zeros_c)
    aggf = _sc_fin(s2, g2, dv)
    return _tc_c(aggf, W2, b2.reshape(1, DO))


# speedup vs baseline: 73.0553x; 1.0083x over previous
import numpy as np

import jax
import jax.numpy as jnp
from jax import lax
from jax.experimental import pallas as pl
from jax.experimental.pallas import tpu as pltpu
from jax.experimental.pallas import tpu_sc as plsc

N = 10000
NP = 10240
E = 320000
EP = 327680
NWORK = 32
CHUNKS = 80
CW = 128
RPT = NP // 16
DH = 16
DI = 128
DO = 128
RB = 1024

_PAD_EDGES = np.stack([
    np.zeros(EP - E, np.int32),
    (N + np.arange(EP - E) % (NP - N)).astype(np.int32),
])


def _mesh():
    return plsc.VectorSubcoreMesh(
        core_axis_name="c", subcore_axis_name="s", num_cores=2, num_subcores=16
    )


_SC_PARAMS = pltpu.CompilerParams(use_tc_tiling_on_sc=False)


def _rsqrt16(x):
    xi = lax.bitcast_convert_type(x, jnp.int32)
    yi = jnp.int32(0x5F3759DF) - (xi >> 1)
    y = lax.bitcast_convert_type(yi, jnp.float32)
    for _ in range(3):
        y = y * (1.5 - 0.5 * x * y * y)
    return y


def _edge_pipeline(srcv, dstv, bufs, gsh, accum, sg, ss):
    nb = len(bufs)
    pltpu.async_copy(gsh.at[srcv.at[0]], bufs[0], sg[0])
    pltpu.async_copy(gsh.at[srcv.at[1]], bufs[1], sg[1])

    @pl.loop(0, CHUNKS, step=nb)
    def _pipe(j0):
        for b in range(nb):
            j = j0 + b
            b2 = (b + 2) % nb

            @pl.when(jnp.logical_and(j + 2 < CHUNKS, j >= 2))
            def _():
                pltpu.make_async_copy(bufs[b2], accum.at[dstv.at[0]], ss[b2]).wait()

            @pl.when(j + 2 < CHUNKS)
            def _():
                pltpu.async_copy(gsh.at[srcv.at[j + 2]], bufs[b2], sg[b2])

            pltpu.make_async_copy(gsh.at[srcv.at[0]], bufs[b], sg[b]).wait()
            pltpu.async_copy(bufs[b], accum.at[dstv.at[j]], ss[b], add=True)

    for b in range(nb):
        pltpu.make_async_copy(bufs[b], accum.at[dstv.at[0]], ss[b]).wait()


def _deg_body(er_hbm, ones_hbm, zeros_hbm, out_hbm, dstv, ones_rows, accum, sem, sem2):
    cid = lax.axis_index("c")
    sid = lax.axis_index("s")
    wid = cid * 16 + sid
    sl = pl.ds(sid * RPT, RPT)

    pltpu.async_copy(ones_hbm, ones_rows, sem2)
    pltpu.sync_copy(zeros_hbm, accum.at[sl])
    pltpu.sync_copy(er_hbm.at[1, wid], dstv)
    pltpu.make_async_copy(ones_hbm, ones_rows, sem2).wait()
    plsc.subcore_barrier()

    def ch(j, _):
        pltpu.async_copy(ones_rows, accum.at[dstv.at[j]], sem, add=True)
        return 0

    lax.fori_loop(0, CHUNKS, ch, 0)

    def dr(j, _):
        pltpu.make_async_copy(ones_rows, accum.at[dstv.at[0]], sem).wait()
        return 0

    lax.fori_loop(0, CHUNKS, dr, 0)
    plsc.subcore_barrier()

    pltpu.sync_copy(accum.at[sl], out_hbm.at[cid, sl])


_sc_deg = pl.kernel(
    _deg_body,
    out_type=jax.ShapeDtypeStruct((2, NP, DH), jnp.float32),
    mesh=_mesh(),
    scratch_types=[
        pltpu.VMEM((CHUNKS, CW), jnp.int32),
        pltpu.VMEM((CW, DH), jnp.float32),
        pltpu.VMEM_SHARED((NP, DH), jnp.float32),
        pltpu.SemaphoreType.DMA,
        pltpu.SemaphoreType.DMA,
    ],
    compiler_params=_SC_PARAMS,
)


def _agg1_body(
    h_hbm, d_hbm, er_hbm, zeros_hbm,
    s1_hbm, g1_hbm, dv_hbm,
    srcv, dstv, r0, r1, r2, r3, vh, vd0, vd1, gsh, accum,
    g0, g1s, g2s, g3, s0, s1s, s2s, s3,
):
    cid = lax.axis_index("c")
    sid = lax.axis_index("s")
    wid = cid * 16 + sid
    sl = pl.ds(sid * RPT, RPT)

    pltpu.async_copy(h_hbm.at[sl], vh, g0)
    pltpu.async_copy(d_hbm.at[0, sl], vd0, g1s)
    pltpu.async_copy(d_hbm.at[1, sl], vd1, g2s)
    pltpu.sync_copy(zeros_hbm, accum.at[sl])
    pltpu.sync_copy(er_hbm.at[0, wid], srcv)
    pltpu.sync_copy(er_hbm.at[1, wid], dstv)
    pltpu.make_async_copy(h_hbm.at[sl], vh, g0).wait()
    pltpu.make_async_copy(d_hbm.at[0, sl], vd0, g1s).wait()
    pltpu.make_async_copy(d_hbm.at[1, sl], vd1, g2s).wait()

    def cb(i, _):
        deg = 1.0 + vd0[i, :] + vd1[i, :]
        dv = _rsqrt16(deg)
        vh[i, :] = vh[i, :] * dv
        vd1[i, :] = dv
        return 0

    lax.fori_loop(0, RPT, cb, 0, unroll=8)

    pltpu.async_copy(vh, g1_hbm.at[sl], s3)
    pltpu.async_copy(vd1, dv_hbm.at[sl], s3)
    pltpu.sync_copy(vh, gsh.at[sl])
    plsc.subcore_barrier()
    pltpu.make_async_copy(vh, g1_hbm.at[sl], s3).wait()
    pltpu.make_async_copy(vd1, dv_hbm.at[sl], s3).wait()
    _edge_pipeline(srcv, dstv, (r0, r1, r2, r3), gsh, accum,
                   (g0, g1s, g2s, g3), (s0, s1s, s2s, s3))
    plsc.subcore_barrier()
    pltpu.sync_copy(accum.at[sl], s1_hbm.at[cid, sl])


_sc_agg1 = pl.kernel(
    _agg1_body,
    out_type=[
        jax.ShapeDtypeStruct((2, NP, DH), jnp.float32),
        jax.ShapeDtypeStruct((NP, DH), jnp.float32),
        jax.ShapeDtypeStruct((NP, DH), jnp.float32),
    ],
    mesh=_mesh(),
    scratch_types=[
        pltpu.VMEM((CHUNKS, CW), jnp.int32),
        pltpu.VMEM((CHUNKS, CW), jnp.int32),
        pltpu.VMEM((CW, DH), jnp.float32),
        pltpu.VMEM((CW, DH), jnp.float32),
        pltpu.VMEM((CW, DH), jnp.float32),
        pltpu.VMEM((CW, DH), jnp.float32),
        pltpu.VMEM((RPT, DH), jnp.float32),
        pltpu.VMEM((RPT, DH), jnp.float32),
        pltpu.VMEM((RPT, DH), jnp.float32),
        pltpu.VMEM_SHARED((NP, DH), jnp.float32),
        pltpu.VMEM_SHARED((NP, DH), jnp.float32),
        pltpu.SemaphoreType.DMA,
        pltpu.SemaphoreType.DMA,
        pltpu.SemaphoreType.DMA,
        pltpu.SemaphoreType.DMA,
        pltpu.SemaphoreType.DMA,
        pltpu.SemaphoreType.DMA,
        pltpu.SemaphoreType.DMA,
        pltpu.SemaphoreType.DMA,
    ],
    compiler_params=_SC_PARAMS,
)


def _agg2_body(
    s1_hbm, g1_hbm, dv_hbm, b1_hbm, er_hbm, zeros_hbm,
    s2_hbm, g2_hbm,
    srcv, dstv, r0, r1, r2, r3, vg, vs0, vs1, vdv, b1v, gsh, accum,
    g0, g1s, g2s, g3, s0, s1s, s2s, s3,
):
    cid = lax.axis_index("c")
    sid = lax.axis_index("s")
    wid = cid * 16 + sid
    sl = pl.ds(sid * RPT, RPT)

    pltpu.async_copy(g1_hbm.at[sl], vg, g0)
    pltpu.async_copy(s1_hbm.at[0, sl], vs0, g1s)
    pltpu.async_copy(s1_hbm.at[1, sl], vs1, g2s)
    pltpu.async_copy(dv_hbm.at[sl], vdv, g3)
    pltpu.sync_copy(zeros_hbm, accum.at[sl])
    pltpu.sync_copy(er_hbm.at[0, wid], srcv)
    pltpu.sync_copy(er_hbm.at[1, wid], dstv)
    pltpu.sync_copy(b1_hbm, b1v)
    b1 = b1v[...]
    pltpu.make_async_copy(g1_hbm.at[sl], vg, g0).wait()
    pltpu.make_async_copy(s1_hbm.at[0, sl], vs0, g1s).wait()
    pltpu.make_async_copy(s1_hbm.at[1, sl], vs1, g2s).wait()
    pltpu.make_async_copy(dv_hbm.at[sl], vdv, g3).wait()

    def cb(i, _):
        dv = vdv[i, :]
        o1 = dv * (vs0[i, :] + vs1[i, :] + vg[i, :]) + b1
        vg[i, :] = jnp.maximum(o1, 0.0) * dv
        return 0

    lax.fori_loop(0, RPT, cb, 0, unroll=8)

    pltpu.async_copy(vg, g2_hbm.at[sl], s3)
    pltpu.sync_copy(vg, gsh.at[sl])
    plsc.subcore_barrier()
    pltpu.make_async_copy(vg, g2_hbm.at[sl], s3).wait()
    _edge_pipeline(srcv, dstv, (r0, r1, r2, r3), gsh, accum,
                   (g0, g1s, g2s, g3), (s0, s1s, s2s, s3))
    plsc.subcore_barrier()
    pltpu.sync_copy(accum.at[sl], s2_hbm.at[cid, sl])


_sc_agg2 = pl.kernel(
    _agg2_body,
    out_type=[
        jax.ShapeDtypeStruct((2, NP, DH), jnp.float32),
        jax.ShapeDtypeStruct((NP, DH), jnp.float32),
    ],
    mesh=_mesh(),
    scratch_types=[
        pltpu.VMEM((CHUNKS, CW), jnp.int32),
        pltpu.VMEM((CHUNKS, CW), jnp.int32),
        pltpu.VMEM((CW, DH), jnp.float32),
        pltpu.VMEM((CW, DH), jnp.float32),
        pltpu.VMEM((CW, DH), jnp.float32),
        pltpu.VMEM((CW, DH), jnp.float32),
        pltpu.VMEM((RPT, DH), jnp.float32),
        pltpu.VMEM((RPT, DH), jnp.float32),
        pltpu.VMEM((RPT, DH), jnp.float32),
        pltpu.VMEM((RPT, DH), jnp.float32),
        pltpu.VMEM((DH,), jnp.float32),
        pltpu.VMEM_SHARED((NP, DH), jnp.float32),
        pltpu.VMEM_SHARED((NP, DH), jnp.float32),
        pltpu.SemaphoreType.DMA,
        pltpu.SemaphoreType.DMA,
        pltpu.SemaphoreType.DMA,
        pltpu.SemaphoreType.DMA,
        pltpu.SemaphoreType.DMA,
        pltpu.SemaphoreType.DMA,
        pltpu.SemaphoreType.DMA,
        pltpu.SemaphoreType.DMA,
    ],
    compiler_params=_SC_PARAMS,
)


def _fin_body(s2_hbm, g2_hbm, dv_hbm, aggf_hbm, vs0, vs1, vg, vdv, g0, g1s, g2s, g3):
    sid = lax.axis_index("s")
    cid = lax.axis_index("c")
    w = sid * 2 + cid
    sl = pl.ds(w * (RPT // 2), RPT // 2)

    pltpu.async_copy(s2_hbm.at[0, sl], vs0, g0)
    pltpu.async_copy(s2_hbm.at[1, sl], vs1, g1s)
    pltpu.async_copy(g2_hbm.at[sl], vg, g2s)
    pltpu.async_copy(dv_hbm.at[sl], vdv, g3)
    pltpu.make_async_copy(s2_hbm.at[0, sl], vs0, g0).wait()
    pltpu.make_async_copy(s2_hbm.at[1, sl], vs1, g1s).wait()
    pltpu.make_async_copy(g2_hbm.at[sl], vg, g2s).wait()
    pltpu.make_async_copy(dv_hbm.at[sl], vdv, g3).wait()

    def cb(i, _):
        vg[i, :] = vdv[i, :] * (vs0[i, :] + vs1[i, :] + vg[i, :])
        return 0

    lax.fori_loop(0, RPT // 2, cb, 0, unroll=8)
    pltpu.sync_copy(vg, aggf_hbm.at[sl])


_sc_fin = pl.kernel(
    _fin_body,
    out_type=jax.ShapeDtypeStruct((NP, DH), jnp.float32),
    mesh=_mesh(),
    scratch_types=[
        pltpu.VMEM((RPT // 2, DH), jnp.float32),
        pltpu.VMEM((RPT // 2, DH), jnp.float32),
        pltpu.VMEM((RPT // 2, DH), jnp.float32),
        pltpu.VMEM((RPT // 2, DH), jnp.float32),
        pltpu.SemaphoreType.DMA,
        pltpu.SemaphoreType.DMA,
        pltpu.SemaphoreType.DMA,
        pltpu.SemaphoreType.DMA,
    ],
    compiler_params=_SC_PARAMS,
)


def _tc_a_body(x_ref, w_ref, h_ref):
    h_ref[...] = jnp.dot(x_ref[...], w_ref[...], preferred_element_type=jnp.float32)


def _tc_a(x, W1):
    return pl.pallas_call(
        _tc_a_body,
        grid=(NP // RB,),
        in_specs=[
            pl.BlockSpec((RB, DI), lambda i: (i, 0)),
            pl.BlockSpec((DI, DH), lambda i: (0, 0)),
        ],
        out_specs=pl.BlockSpec((RB, DH), lambda i: (i, 0)),
        out_shape=jax.ShapeDtypeStruct((NP, DH), jnp.float32),
    )(x, W1)


def _tc_c_body(a_ref, w_ref, b_ref, o_ref):
    o_ref[...] = (
        jnp.dot(a_ref[...], w_ref[...], preferred_element_type=jnp.float32)
        + b_ref[...]
    )


def _tc_c(aggf, W2, b2):
    return pl.pallas_call(
        _tc_c_body,
        grid=(NP // RB,),
        in_specs=[
            pl.BlockSpec((RB, DH), lambda i: (i, 0)),
            pl.BlockSpec((DH, DO), lambda i: (0, 0)),
            pl.BlockSpec((1, DO), lambda i: (0, 0)),
        ],
        out_specs=pl.BlockSpec((RB, DO), lambda i: (i, 0)),
        out_shape=jax.ShapeDtypeStruct((N, DO), jnp.float32),
    )(aggf, W2, b2)


def kernel(x, edge_index, W1, b1, W2, b2):
    er = jnp.concatenate(
        [edge_index, jnp.asarray(_PAD_EDGES)], axis=1
    ).reshape(2, NWORK, CHUNKS, CW)
    zeros_c = jnp.zeros((RPT, DH), jnp.float32)
    ones_c = jnp.ones((CW, DH), jnp.float32)

    h = _tc_a(x, W1)
    degp = _sc_deg(er, ones_c, zeros_c)
    s1, g1, dv = _sc_agg1(h, degp, er, zeros_c)
    s2, g2 = _sc_agg2(s1, g1, dv, b1, er, zeros_c)
    aggf = _sc_fin(s2, g2, dv)
    return _tc_c(aggf, W2, b2.reshape(1, DO))


# speedup vs baseline: 75.7595x; 1.0370x over previous
import numpy as np

import jax
import jax.numpy as jnp
from jax import lax
from jax.experimental import pallas as pl
from jax.experimental.pallas import tpu as pltpu
from jax.experimental.pallas import tpu_sc as plsc

N = 10000
NP = 10240
E = 320000
EP = 327680
NWORK = 32
CHUNKS = 80
CW = 128
RPT = NP // 16
DH = 16
DI = 128
DO = 128
RB = 1024

_PAD_EDGES = np.stack([
    (N + (np.arange(EP - E) * 7 + 3) % (NP - N)).astype(np.int32),
    (N + np.arange(EP - E) % (NP - N)).astype(np.int32),
])


def _mesh():
    return plsc.VectorSubcoreMesh(
        core_axis_name="c", subcore_axis_name="s", num_cores=2, num_subcores=16
    )


_SC_PARAMS = pltpu.CompilerParams(use_tc_tiling_on_sc=False)


def _rsqrt16(x):
    xi = lax.bitcast_convert_type(x, jnp.int32)
    yi = jnp.int32(0x5F3759DF) - (xi >> 1)
    y = lax.bitcast_convert_type(yi, jnp.float32)
    for _ in range(3):
        y = y * (1.5 - 0.5 * x * y * y)
    return y


def _edge_pipeline(srcv, dstv, bufs, gsh, accum, sg, ss):
    nb = len(bufs)
    pltpu.async_copy(gsh.at[srcv.at[0]], bufs[0], sg[0])
    pltpu.async_copy(gsh.at[srcv.at[1]], bufs[1], sg[1])

    @pl.loop(0, CHUNKS, step=nb)
    def _pipe(j0):
        for b in range(nb):
            j = j0 + b
            b2 = (b + 2) % nb

            @pl.when(jnp.logical_and(j + 2 < CHUNKS, j >= 2))
            def _():
                pltpu.make_async_copy(bufs[b2], accum.at[dstv.at[0]], ss[b2]).wait()

            @pl.when(j + 2 < CHUNKS)
            def _():
                pltpu.async_copy(gsh.at[srcv.at[j + 2]], bufs[b2], sg[b2])

            pltpu.make_async_copy(gsh.at[srcv.at[0]], bufs[b], sg[b]).wait()
            pltpu.async_copy(bufs[b], accum.at[dstv.at[j]], ss[b], add=True)

    for b in range(nb):
        pltpu.make_async_copy(bufs[b], accum.at[dstv.at[0]], ss[b]).wait()


def _deg_body(er_hbm, ones_hbm, zeros_hbm, out_hbm, dstv, ones_rows, accum, sem, sem2):
    cid = lax.axis_index("c")
    sid = lax.axis_index("s")
    wid = cid * 16 + sid
    sl = pl.ds(sid * RPT, RPT)

    pltpu.async_copy(ones_hbm, ones_rows, sem2)
    pltpu.sync_copy(zeros_hbm, accum.at[sl])
    pltpu.sync_copy(er_hbm.at[1, wid], dstv)
    pltpu.make_async_copy(ones_hbm, ones_rows, sem2).wait()
    plsc.subcore_barrier()

    def ch(j, _):
        pltpu.async_copy(ones_rows, accum.at[dstv.at[j]], sem, add=True)
        return 0

    lax.fori_loop(0, CHUNKS, ch, 0)

    def dr(j, _):
        pltpu.make_async_copy(ones_rows, accum.at[dstv.at[0]], sem).wait()
        return 0

    lax.fori_loop(0, CHUNKS, dr, 0)
    plsc.subcore_barrier()

    pltpu.sync_copy(accum.at[sl], out_hbm.at[cid, sl])


_sc_deg = pl.kernel(
    _deg_body,
    out_type=jax.ShapeDtypeStruct((2, NP, DH), jnp.float32),
    mesh=_mesh(),
    scratch_types=[
        pltpu.VMEM((CHUNKS, CW), jnp.int32),
        pltpu.VMEM((CW, DH), jnp.float32),
        pltpu.VMEM_SHARED((NP, DH), jnp.float32),
        pltpu.SemaphoreType.DMA,
        pltpu.SemaphoreType.DMA,
    ],
    compiler_params=_SC_PARAMS,
)


def _agg1_body(
    h_hbm, d_hbm, er_hbm, zeros_hbm,
    s1_hbm, g1_hbm, dv_hbm,
    srcv, dstv, r0, r1, r2, r3, vh, vd0, vd1, gsh, accum,
    g0, g1s, g2s, g3, s0, s1s, s2s, s3,
):
    cid = lax.axis_index("c")
    sid = lax.axis_index("s")
    wid = cid * 16 + sid
    sl = pl.ds(sid * RPT, RPT)

    pltpu.async_copy(h_hbm.at[sl], vh, g0)
    pltpu.async_copy(d_hbm.at[0, sl], vd0, g1s)
    pltpu.async_copy(d_hbm.at[1, sl], vd1, g2s)
    pltpu.sync_copy(zeros_hbm, accum.at[sl])
    pltpu.sync_copy(er_hbm.at[0, wid], srcv)
    pltpu.sync_copy(er_hbm.at[1, wid], dstv)
    pltpu.make_async_copy(h_hbm.at[sl], vh, g0).wait()
    pltpu.make_async_copy(d_hbm.at[0, sl], vd0, g1s).wait()
    pltpu.make_async_copy(d_hbm.at[1, sl], vd1, g2s).wait()

    def cb(i, _):
        deg = 1.0 + vd0[i, :] + vd1[i, :]
        dv = _rsqrt16(deg)
        vh[i, :] = vh[i, :] * dv
        vd1[i, :] = dv
        return 0

    lax.fori_loop(0, RPT, cb, 0, unroll=8)

    pltpu.async_copy(vh, g1_hbm.at[sl], s3)
    pltpu.async_copy(vd1, dv_hbm.at[sl], s3)
    pltpu.sync_copy(vh, gsh.at[sl])
    plsc.subcore_barrier()
    pltpu.make_async_copy(vh, g1_hbm.at[sl], s3).wait()
    pltpu.make_async_copy(vd1, dv_hbm.at[sl], s3).wait()
    _edge_pipeline(srcv, dstv, (r0, r1, r2, r3), gsh, accum,
                   (g0, g1s, g2s, g3), (s0, s1s, s2s, s3))
    plsc.subcore_barrier()
    pltpu.sync_copy(accum.at[sl], s1_hbm.at[cid, sl])


_sc_agg1 = pl.kernel(
    _agg1_body,
    out_type=[
        jax.ShapeDtypeStruct((2, NP, DH), jnp.float32),
        jax.ShapeDtypeStruct((NP, DH), jnp.float32),
        jax.ShapeDtypeStruct((NP, DH), jnp.float32),
    ],
    mesh=_mesh(),
    scratch_types=[
        pltpu.VMEM((CHUNKS, CW), jnp.int32),
        pltpu.VMEM((CHUNKS, CW), jnp.int32),
        pltpu.VMEM((CW, DH), jnp.float32),
        pltpu.VMEM((CW, DH), jnp.float32),
        pltpu.VMEM((CW, DH), jnp.float32),
        pltpu.VMEM((CW, DH), jnp.float32),
        pltpu.VMEM((RPT, DH), jnp.float32),
        pltpu.VMEM((RPT, DH), jnp.float32),
        pltpu.VMEM((RPT, DH), jnp.float32),
        pltpu.VMEM_SHARED((NP, DH), jnp.float32),
        pltpu.VMEM_SHARED((NP, DH), jnp.float32),
        pltpu.SemaphoreType.DMA,
        pltpu.SemaphoreType.DMA,
        pltpu.SemaphoreType.DMA,
        pltpu.SemaphoreType.DMA,
        pltpu.SemaphoreType.DMA,
        pltpu.SemaphoreType.DMA,
        pltpu.SemaphoreType.DMA,
        pltpu.SemaphoreType.DMA,
    ],
    compiler_params=_SC_PARAMS,
)


def _agg2_body(
    s1_hbm, g1_hbm, dv_hbm, b1_hbm, er_hbm, zeros_hbm,
    s2_hbm, g2_hbm,
    srcv, dstv, r0, r1, r2, r3, vg, vs0, vs1, vdv, b1v, gsh, accum,
    g0, g1s, g2s, g3, s0, s1s, s2s, s3,
):
    cid = lax.axis_index("c")
    sid = lax.axis_index("s")
    wid = cid * 16 + sid
    sl = pl.ds(sid * RPT, RPT)

    pltpu.async_copy(g1_hbm.at[sl], vg, g0)
    pltpu.async_copy(s1_hbm.at[0, sl], vs0, g1s)
    pltpu.async_copy(s1_hbm.at[1, sl], vs1, g2s)
    pltpu.async_copy(dv_hbm.at[sl], vdv, g3)
    pltpu.sync_copy(zeros_hbm, accum.at[sl])
    pltpu.sync_copy(er_hbm.at[0, wid], srcv)
    pltpu.sync_copy(er_hbm.at[1, wid], dstv)
    pltpu.sync_copy(b1_hbm, b1v)
    b1 = b1v[...]
    pltpu.make_async_copy(g1_hbm.at[sl], vg, g0).wait()
    pltpu.make_async_copy(s1_hbm.at[0, sl], vs0, g1s).wait()
    pltpu.make_async_copy(s1_hbm.at[1, sl], vs1, g2s).wait()
    pltpu.make_async_copy(dv_hbm.at[sl], vdv, g3).wait()

    def cb(i, _):
        dv = vdv[i, :]
        o1 = dv * (vs0[i, :] + vs1[i, :] + vg[i, :]) + b1
        vg[i, :] = jnp.maximum(o1, 0.0) * dv
        return 0

    lax.fori_loop(0, RPT, cb, 0, unroll=8)

    pltpu.async_copy(vg, g2_hbm.at[sl], s3)
    pltpu.sync_copy(vg, gsh.at[sl])
    plsc.subcore_barrier()
    pltpu.make_async_copy(vg, g2_hbm.at[sl], s3).wait()
    _edge_pipeline(srcv, dstv, (r0, r1, r2, r3), gsh, accum,
                   (g0, g1s, g2s, g3), (s0, s1s, s2s, s3))
    plsc.subcore_barrier()
    pltpu.sync_copy(accum.at[sl], s2_hbm.at[cid, sl])


_sc_agg2 = pl.kernel(
    _agg2_body,
    out_type=[
        jax.ShapeDtypeStruct((2, NP, DH), jnp.float32),
        jax.ShapeDtypeStruct((NP, DH), jnp.float32),
    ],
    mesh=_mesh(),
    scratch_types=[
        pltpu.VMEM((CHUNKS, CW), jnp.int32),
        pltpu.VMEM((CHUNKS, CW), jnp.int32),
        pltpu.VMEM((CW, DH), jnp.float32),
        pltpu.VMEM((CW, DH), jnp.float32),
        pltpu.VMEM((CW, DH), jnp.float32),
        pltpu.VMEM((CW, DH), jnp.float32),
        pltpu.VMEM((RPT, DH), jnp.float32),
        pltpu.VMEM((RPT, DH), jnp.float32),
        pltpu.VMEM((RPT, DH), jnp.float32),
        pltpu.VMEM((RPT, DH), jnp.float32),
        pltpu.VMEM((DH,), jnp.float32),
        pltpu.VMEM_SHARED((NP, DH), jnp.float32),
        pltpu.VMEM_SHARED((NP, DH), jnp.float32),
        pltpu.SemaphoreType.DMA,
        pltpu.SemaphoreType.DMA,
        pltpu.SemaphoreType.DMA,
        pltpu.SemaphoreType.DMA,
        pltpu.SemaphoreType.DMA,
        pltpu.SemaphoreType.DMA,
        pltpu.SemaphoreType.DMA,
        pltpu.SemaphoreType.DMA,
    ],
    compiler_params=_SC_PARAMS,
)


def _fin_body(s2_hbm, g2_hbm, dv_hbm, aggf_hbm, vs0, vs1, vg, vdv, g0, g1s, g2s, g3):
    sid = lax.axis_index("s")
    cid = lax.axis_index("c")
    w = sid * 2 + cid
    sl = pl.ds(w * (RPT // 2), RPT // 2)

    pltpu.async_copy(s2_hbm.at[0, sl], vs0, g0)
    pltpu.async_copy(s2_hbm.at[1, sl], vs1, g1s)
    pltpu.async_copy(g2_hbm.at[sl], vg, g2s)
    pltpu.async_copy(dv_hbm.at[sl], vdv, g3)
    pltpu.make_async_copy(s2_hbm.at[0, sl], vs0, g0).wait()
    pltpu.make_async_copy(s2_hbm.at[1, sl], vs1, g1s).wait()
    pltpu.make_async_copy(g2_hbm.at[sl], vg, g2s).wait()
    pltpu.make_async_copy(dv_hbm.at[sl], vdv, g3).wait()

    def cb(i, _):
        vg[i, :] = vdv[i, :] * (vs0[i, :] + vs1[i, :] + vg[i, :])
        return 0

    lax.fori_loop(0, RPT // 2, cb, 0, unroll=8)
    pltpu.sync_copy(vg, aggf_hbm.at[sl])


_sc_fin = pl.kernel(
    _fin_body,
    out_type=jax.ShapeDtypeStruct((NP, DH), jnp.float32),
    mesh=_mesh(),
    scratch_types=[
        pltpu.VMEM((RPT // 2, DH), jnp.float32),
        pltpu.VMEM((RPT // 2, DH), jnp.float32),
        pltpu.VMEM((RPT // 2, DH), jnp.float32),
        pltpu.VMEM((RPT // 2, DH), jnp.float32),
        pltpu.SemaphoreType.DMA,
        pltpu.SemaphoreType.DMA,
        pltpu.SemaphoreType.DMA,
        pltpu.SemaphoreType.DMA,
    ],
    compiler_params=_SC_PARAMS,
)


def _tc_a_body(x_ref, w_ref, h_ref):
    h_ref[...] = jnp.dot(x_ref[...], w_ref[...], preferred_element_type=jnp.float32)


def _tc_a(x, W1):
    return pl.pallas_call(
        _tc_a_body,
        grid=(NP // RB,),
        in_specs=[
            pl.BlockSpec((RB, DI), lambda i: (i, 0)),
            pl.BlockSpec((DI, DH), lambda i: (0, 0)),
        ],
        out_specs=pl.BlockSpec((RB, DH), lambda i: (i, 0)),
        out_shape=jax.ShapeDtypeStruct((NP, DH), jnp.float32),
    )(x, W1)


def _tc_c_body(a_ref, w_ref, b_ref, o_ref):
    o_ref[...] = (
        jnp.dot(a_ref[...], w_ref[...], preferred_element_type=jnp.float32)
        + b_ref[...]
    )


def _tc_c(aggf, W2, b2):
    return pl.pallas_call(
        _tc_c_body,
        grid=(NP // RB,),
        in_specs=[
            pl.BlockSpec((RB, DH), lambda i: (i, 0)),
            pl.BlockSpec((DH, DO), lambda i: (0, 0)),
            pl.BlockSpec((1, DO), lambda i: (0, 0)),
        ],
        out_specs=pl.BlockSpec((RB, DO), lambda i: (i, 0)),
        out_shape=jax.ShapeDtypeStruct((N, DO), jnp.float32),
    )(aggf, W2, b2)


def kernel(x, edge_index, W1, b1, W2, b2):
    er = jnp.concatenate(
        [edge_index, jnp.asarray(_PAD_EDGES)], axis=1
    ).reshape(2, NWORK, CHUNKS, CW)
    zeros_c = jnp.zeros((RPT, DH), jnp.float32)
    ones_c = jnp.ones((CW, DH), jnp.float32)

    h = _tc_a(x, W1)
    degp = _sc_deg(er, ones_c, zeros_c)
    s1, g1, dv = _sc_agg1(h, degp, er, zeros_c)
    s2, g2 = _sc_agg2(s1, g1, dv, b1, er, zeros_c)
    aggf = _sc_fin(s2, g2, dv)
    return _tc_c(aggf, W2, b2.reshape(1, DO))


# speedup vs baseline: 77.8482x; 1.0276x over previous
import numpy as np

import jax
import jax.numpy as jnp
from jax import lax
from jax.experimental import pallas as pl
from jax.experimental.pallas import tpu as pltpu
from jax.experimental.pallas import tpu_sc as plsc

N = 10000
NP = 10240
E = 320000
EP = 327680
NWORK = 32
CHUNKS = 80
CW = 128
RPT = NP // 16
DH = 16
DI = 128
DO = 128
RB = 2048

_PAD_EDGES = np.stack([
    (N + (np.arange(EP - E) * 7 + 3) % (NP - N)).astype(np.int32),
    (N + np.arange(EP - E) % (NP - N)).astype(np.int32),
])


def _mesh():
    return plsc.VectorSubcoreMesh(
        core_axis_name="c", subcore_axis_name="s", num_cores=2, num_subcores=16
    )


_SC_PARAMS = pltpu.CompilerParams(use_tc_tiling_on_sc=False)


def _rsqrt16(x):
    xi = lax.bitcast_convert_type(x, jnp.int32)
    yi = jnp.int32(0x5F3759DF) - (xi >> 1)
    y = lax.bitcast_convert_type(yi, jnp.float32)
    for _ in range(3):
        y = y * (1.5 - 0.5 * x * y * y)
    return y


def _edge_pipeline(srcv, dstv, bufs, gsh, accum, sg, ss):
    nb = len(bufs)
    pltpu.async_copy(gsh.at[srcv.at[0]], bufs[0], sg[0])
    pltpu.async_copy(gsh.at[srcv.at[1]], bufs[1], sg[1])

    @pl.loop(0, CHUNKS, step=nb)
    def _pipe(j0):
        for b in range(nb):
            j = j0 + b
            b2 = (b + 2) % nb

            @pl.when(jnp.logical_and(j + 2 < CHUNKS, j >= 2))
            def _():
                pltpu.make_async_copy(bufs[b2], accum.at[dstv.at[0]], ss[b2]).wait()

            @pl.when(j + 2 < CHUNKS)
            def _():
                pltpu.async_copy(gsh.at[srcv.at[j + 2]], bufs[b2], sg[b2])

            pltpu.make_async_copy(gsh.at[srcv.at[0]], bufs[b], sg[b]).wait()
            pltpu.async_copy(bufs[b], accum.at[dstv.at[j]], ss[b], add=True)

    for b in range(nb):
        pltpu.make_async_copy(bufs[b], accum.at[dstv.at[0]], ss[b]).wait()


def _deg_body(er_hbm, ones_hbm, zeros_hbm, out_hbm, dstv, ones_rows, accum, sem, sem2):
    cid = lax.axis_index("c")
    sid = lax.axis_index("s")
    wid = cid * 16 + sid
    sl = pl.ds(sid * RPT, RPT)

    pltpu.async_copy(ones_hbm, ones_rows, sem2)
    pltpu.sync_copy(zeros_hbm, accum.at[sl])
    pltpu.sync_copy(er_hbm.at[1, wid], dstv)
    pltpu.make_async_copy(ones_hbm, ones_rows, sem2).wait()
    plsc.subcore_barrier()

    def ch(j, _):
        pltpu.async_copy(ones_rows, accum.at[dstv.at[j]], sem, add=True)
        return 0

    lax.fori_loop(0, CHUNKS, ch, 0)

    def dr(j, _):
        pltpu.make_async_copy(ones_rows, accum.at[dstv.at[0]], sem).wait()
        return 0

    lax.fori_loop(0, CHUNKS, dr, 0)
    plsc.subcore_barrier()

    pltpu.sync_copy(accum.at[sl], out_hbm.at[cid, sl])


_sc_deg = pl.kernel(
    _deg_body,
    out_type=jax.ShapeDtypeStruct((2, NP, DH), jnp.float32),
    mesh=_mesh(),
    scratch_types=[
        pltpu.VMEM((CHUNKS, CW), jnp.int32),
        pltpu.VMEM((CW, DH), jnp.float32),
        pltpu.VMEM_SHARED((NP, DH), jnp.float32),
        pltpu.SemaphoreType.DMA,
        pltpu.SemaphoreType.DMA,
    ],
    compiler_params=_SC_PARAMS,
)


def _agg1_body(
    h_hbm, d_hbm, er_hbm, zeros_hbm,
    s1_hbm, g1_hbm, dv_hbm,
    srcv, dstv, r0, r1, r2, r3, vh, vd0, vd1, gsh, accum,
    g0, g1s, g2s, g3, s0, s1s, s2s, s3,
):
    cid = lax.axis_index("c")
    sid = lax.axis_index("s")
    wid = cid * 16 + sid
    sl = pl.ds(sid * RPT, RPT)

    pltpu.async_copy(h_hbm.at[sl], vh, g0)
    pltpu.async_copy(d_hbm.at[0, sl], vd0, g1s)
    pltpu.async_copy(d_hbm.at[1, sl], vd1, g2s)
    pltpu.sync_copy(zeros_hbm, accum.at[sl])
    pltpu.sync_copy(er_hbm.at[0, wid], srcv)
    pltpu.sync_copy(er_hbm.at[1, wid], dstv)
    pltpu.make_async_copy(h_hbm.at[sl], vh, g0).wait()
    pltpu.make_async_copy(d_hbm.at[0, sl], vd0, g1s).wait()
    pltpu.make_async_copy(d_hbm.at[1, sl], vd1, g2s).wait()

    def cb(i, _):
        deg = 1.0 + vd0[i, :] + vd1[i, :]
        dv = _rsqrt16(deg)
        vh[i, :] = vh[i, :] * dv
        vd1[i, :] = dv
        return 0

    lax.fori_loop(0, RPT, cb, 0, unroll=8)

    pltpu.async_copy(vh, g1_hbm.at[sl], s3)
    pltpu.async_copy(vd1, dv_hbm.at[sl], s3)
    pltpu.sync_copy(vh, gsh.at[sl])
    plsc.subcore_barrier()
    pltpu.make_async_copy(vh, g1_hbm.at[sl], s3).wait()
    pltpu.make_async_copy(vd1, dv_hbm.at[sl], s3).wait()
    _edge_pipeline(srcv, dstv, (r0, r1, r2, r3), gsh, accum,
                   (g0, g1s, g2s, g3), (s0, s1s, s2s, s3))
    plsc.subcore_barrier()
    pltpu.sync_copy(accum.at[sl], s1_hbm.at[cid, sl])


_sc_agg1 = pl.kernel(
    _agg1_body,
    out_type=[
        jax.ShapeDtypeStruct((2, NP, DH), jnp.float32),
        jax.ShapeDtypeStruct((NP, DH), jnp.float32),
        jax.ShapeDtypeStruct((NP, DH), jnp.float32),
    ],
    mesh=_mesh(),
    scratch_types=[
        pltpu.VMEM((CHUNKS, CW), jnp.int32),
        pltpu.VMEM((CHUNKS, CW), jnp.int32),
        pltpu.VMEM((CW, DH), jnp.float32),
        pltpu.VMEM((CW, DH), jnp.float32),
        pltpu.VMEM((CW, DH), jnp.float32),
        pltpu.VMEM((CW, DH), jnp.float32),
        pltpu.VMEM((RPT, DH), jnp.float32),
        pltpu.VMEM((RPT, DH), jnp.float32),
        pltpu.VMEM((RPT, DH), jnp.float32),
        pltpu.VMEM_SHARED((NP, DH), jnp.float32),
        pltpu.VMEM_SHARED((NP, DH), jnp.float32),
        pltpu.SemaphoreType.DMA,
        pltpu.SemaphoreType.DMA,
        pltpu.SemaphoreType.DMA,
        pltpu.SemaphoreType.DMA,
        pltpu.SemaphoreType.DMA,
        pltpu.SemaphoreType.DMA,
        pltpu.SemaphoreType.DMA,
        pltpu.SemaphoreType.DMA,
    ],
    compiler_params=_SC_PARAMS,
)


def _agg2_body(
    s1_hbm, g1_hbm, dv_hbm, b1_hbm, er_hbm, zeros_hbm,
    s2_hbm, g2_hbm,
    srcv, dstv, r0, r1, r2, r3, vg, vs0, vs1, vdv, b1v, gsh, accum,
    g0, g1s, g2s, g3, s0, s1s, s2s, s3,
):
    cid = lax.axis_index("c")
    sid = lax.axis_index("s")
    wid = cid * 16 + sid
    sl = pl.ds(sid * RPT, RPT)

    pltpu.async_copy(g1_hbm.at[sl], vg, g0)
    pltpu.async_copy(s1_hbm.at[0, sl], vs0, g1s)
    pltpu.async_copy(s1_hbm.at[1, sl], vs1, g2s)
    pltpu.async_copy(dv_hbm.at[sl], vdv, g3)
    pltpu.sync_copy(zeros_hbm, accum.at[sl])
    pltpu.sync_copy(er_hbm.at[0, wid], srcv)
    pltpu.sync_copy(er_hbm.at[1, wid], dstv)
    pltpu.sync_copy(b1_hbm, b1v)
    b1 = b1v[...]
    pltpu.make_async_copy(g1_hbm.at[sl], vg, g0).wait()
    pltpu.make_async_copy(s1_hbm.at[0, sl], vs0, g1s).wait()
    pltpu.make_async_copy(s1_hbm.at[1, sl], vs1, g2s).wait()
    pltpu.make_async_copy(dv_hbm.at[sl], vdv, g3).wait()

    def cb(i, _):
        dv = vdv[i, :]
        o1 = dv * (vs0[i, :] + vs1[i, :] + vg[i, :]) + b1
        vg[i, :] = jnp.maximum(o1, 0.0) * dv
        return 0

    lax.fori_loop(0, RPT, cb, 0, unroll=8)

    pltpu.async_copy(vg, g2_hbm.at[sl], s3)
    pltpu.sync_copy(vg, gsh.at[sl])
    plsc.subcore_barrier()
    pltpu.make_async_copy(vg, g2_hbm.at[sl], s3).wait()
    _edge_pipeline(srcv, dstv, (r0, r1, r2, r3), gsh, accum,
                   (g0, g1s, g2s, g3), (s0, s1s, s2s, s3))
    plsc.subcore_barrier()
    pltpu.sync_copy(accum.at[sl], s2_hbm.at[cid, sl])


_sc_agg2 = pl.kernel(
    _agg2_body,
    out_type=[
        jax.ShapeDtypeStruct((2, NP, DH), jnp.float32),
        jax.ShapeDtypeStruct((NP, DH), jnp.float32),
    ],
    mesh=_mesh(),
    scratch_types=[
        pltpu.VMEM((CHUNKS, CW), jnp.int32),
        pltpu.VMEM((CHUNKS, CW), jnp.int32),
        pltpu.VMEM((CW, DH), jnp.float32),
        pltpu.VMEM((CW, DH), jnp.float32),
        pltpu.VMEM((CW, DH), jnp.float32),
        pltpu.VMEM((CW, DH), jnp.float32),
        pltpu.VMEM((RPT, DH), jnp.float32),
        pltpu.VMEM((RPT, DH), jnp.float32),
        pltpu.VMEM((RPT, DH), jnp.float32),
        pltpu.VMEM((RPT, DH), jnp.float32),
        pltpu.VMEM((DH,), jnp.float32),
        pltpu.VMEM_SHARED((NP, DH), jnp.float32),
        pltpu.VMEM_SHARED((NP, DH), jnp.float32),
        pltpu.SemaphoreType.DMA,
        pltpu.SemaphoreType.DMA,
        pltpu.SemaphoreType.DMA,
        pltpu.SemaphoreType.DMA,
        pltpu.SemaphoreType.DMA,
        pltpu.SemaphoreType.DMA,
        pltpu.SemaphoreType.DMA,
        pltpu.SemaphoreType.DMA,
    ],
    compiler_params=_SC_PARAMS,
)


def _fin_body(s2_hbm, g2_hbm, dv_hbm, aggf_hbm, vs0, vs1, vg, vdv, g0, g1s, g2s, g3):
    sid = lax.axis_index("s")
    cid = lax.axis_index("c")
    w = sid * 2 + cid
    sl = pl.ds(w * (RPT // 2), RPT // 2)

    pltpu.async_copy(s2_hbm.at[0, sl], vs0, g0)
    pltpu.async_copy(s2_hbm.at[1, sl], vs1, g1s)
    pltpu.async_copy(g2_hbm.at[sl], vg, g2s)
    pltpu.async_copy(dv_hbm.at[sl], vdv, g3)
    pltpu.make_async_copy(s2_hbm.at[0, sl], vs0, g0).wait()
    pltpu.make_async_copy(s2_hbm.at[1, sl], vs1, g1s).wait()
    pltpu.make_async_copy(g2_hbm.at[sl], vg, g2s).wait()
    pltpu.make_async_copy(dv_hbm.at[sl], vdv, g3).wait()

    def cb(i, _):
        vg[i, :] = vdv[i, :] * (vs0[i, :] + vs1[i, :] + vg[i, :])
        return 0

    lax.fori_loop(0, RPT // 2, cb, 0, unroll=8)
    pltpu.sync_copy(vg, aggf_hbm.at[sl])


_sc_fin = pl.kernel(
    _fin_body,
    out_type=jax.ShapeDtypeStruct((NP, DH), jnp.float32),
    mesh=_mesh(),
    scratch_types=[
        pltpu.VMEM((RPT // 2, DH), jnp.float32),
        pltpu.VMEM((RPT // 2, DH), jnp.float32),
        pltpu.VMEM((RPT // 2, DH), jnp.float32),
        pltpu.VMEM((RPT // 2, DH), jnp.float32),
        pltpu.SemaphoreType.DMA,
        pltpu.SemaphoreType.DMA,
        pltpu.SemaphoreType.DMA,
        pltpu.SemaphoreType.DMA,
    ],
    compiler_params=_SC_PARAMS,
)


def _tc_a_body(x_ref, w_ref, h_ref):
    h_ref[...] = jnp.dot(x_ref[...], w_ref[...], preferred_element_type=jnp.float32)


def _tc_a(x, W1):
    return pl.pallas_call(
        _tc_a_body,
        grid=(NP // RB,),
        in_specs=[
            pl.BlockSpec((RB, DI), lambda i: (i, 0)),
            pl.BlockSpec((DI, DH), lambda i: (0, 0)),
        ],
        out_specs=pl.BlockSpec((RB, DH), lambda i: (i, 0)),
        out_shape=jax.ShapeDtypeStruct((NP, DH), jnp.float32),
    )(x, W1)


def _tc_c_body(a_ref, w_ref, b_ref, o_ref):
    o_ref[...] = (
        jnp.dot(a_ref[...], w_ref[...], preferred_element_type=jnp.float32)
        + b_ref[...]
    )


def _tc_c(aggf, W2, b2):
    return pl.pallas_call(
        _tc_c_body,
        grid=(NP // RB,),
        in_specs=[
            pl.BlockSpec((RB, DH), lambda i: (i, 0)),
            pl.BlockSpec((DH, DO), lambda i: (0, 0)),
            pl.BlockSpec((1, DO), lambda i: (0, 0)),
        ],
        out_specs=pl.BlockSpec((RB, DO), lambda i: (i, 0)),
        out_shape=jax.ShapeDtypeStruct((N, DO), jnp.float32),
    )(aggf, W2, b2)


def kernel(x, edge_index, W1, b1, W2, b2):
    er = jnp.concatenate(
        [edge_index, jnp.asarray(_PAD_EDGES)], axis=1
    ).reshape(2, NWORK, CHUNKS, CW)
    zeros_c = jnp.zeros((RPT, DH), jnp.float32)
    ones_c = jnp.ones((CW, DH), jnp.float32)

    h = _tc_a(x, W1)
    degp = _sc_deg(er, ones_c, zeros_c)
    s1, g1, dv = _sc_agg1(h, degp, er, zeros_c)
    s2, g2 = _sc_agg2(s1, g1, dv, b1, er, zeros_c)
    aggf = _sc_fin(s2, g2, dv)
    return _tc_c(aggf, W2, b2.reshape(1, DO))
